# baseline (speedup 1.0000x reference)
import functools

import jax
import jax.numpy as jnp
import numpy as np
from jax import lax
from jax.experimental import pallas as pl
from jax.experimental.pallas import tpu as pltpu
from jax.experimental.pallas import tpu_sc as plsc

N_ATOMS = 10000
N_EDGES = 160000
N_TRIPLETS = 160000
F = 128
R = 6
S = 7
CUTOFF = 5.0
P = 7

NC, NS, L = 2, 16, 16
NW = NC * NS
EPW = N_EDGES // NW
TPW = N_TRIPLETS // NW
TPS = N_TRIPLETS // NS

def _sc_mesh():
  return plsc.VectorSubcoreMesh(
      core_axis_name="c", subcore_axis_name="s", num_cores=NC, num_subcores=NS)

f32 = jnp.float32
i32 = jnp.int32


def _swish(x):
  return x * jax.nn.sigmoid(x)


def _envelope(x):
  p = P
  a = -(p + 1) * (p + 2) / 2.0
  b = p * (p + 2)
  c = -p * (p + 1) / 2.0
  xp0 = x ** (p - 1)
  xp1 = xp0 * x
  xp2 = xp1 * x
  return 1.0 / x + a * xp0 + b * xp1 + c * xp2


def _wid():
  return lax.axis_index("c") * NS + lax.axis_index("s")


def _sc_edge_d2(pxa, pya, pza, i_arr, j_arr):
  @functools.partial(
      pl.kernel,
      out_type=jax.ShapeDtypeStruct((N_EDGES,), f32),
      mesh=_sc_mesh(),
      compiler_params=pltpu.CompilerParams(needs_layout_passes=False),
      scratch_types=[
          pltpu.VMEM((N_ATOMS,), f32),
          pltpu.VMEM((N_ATOMS,), f32),
          pltpu.VMEM((N_ATOMS,), f32),
          pltpu.VMEM((EPW,), i32),
          pltpu.VMEM((EPW,), i32),
          pltpu.VMEM((EPW,), f32),
      ],
  )
  def k(px_hbm, py_hbm, pz_hbm, i_hbm, j_hbm, d2_hbm, px, py, pz, iv, jv, ov):
    w = _wid()
    base = w * EPW
    pltpu.sync_copy(px_hbm, px)
    pltpu.sync_copy(py_hbm, py)
    pltpu.sync_copy(pz_hbm, pz)
    pltpu.sync_copy(i_hbm.at[pl.ds(base, EPW)], iv)
    pltpu.sync_copy(j_hbm.at[pl.ds(base, EPW)], jv)

    def body(g, _):
      st = jnp.minimum(g * L, EPW - L)
      a16 = iv[pl.ds(st, L)]
      b16 = jv[pl.ds(st, L)]
      dx = plsc.load_gather(px, [a16]) - plsc.load_gather(px, [b16])
      dy = plsc.load_gather(py, [a16]) - plsc.load_gather(py, [b16])
      dz = plsc.load_gather(pz, [a16]) - plsc.load_gather(pz, [b16])
      ov[pl.ds(st, L)] = dx * dx + dy * dy + dz * dz
      return 0

    lax.fori_loop(0, (EPW + L - 1) // L, body, 0)
    pltpu.sync_copy(ov, d2_hbm.at[pl.ds(base, EPW)])

  return k(pxa, pya, pza, i_arr, j_arr)


def _sc_triplet_geom(pxa, pya, pza, i_arr, j_arr, idx_kj, idx_ji, d2):
  out_t = tuple(jax.ShapeDtypeStruct((N_TRIPLETS,), f32) for _ in range(4))

  @functools.partial(
      pl.kernel,
      out_type=out_t,
      mesh=_sc_mesh(),
      compiler_params=pltpu.CompilerParams(needs_layout_passes=False),
      scratch_types=[
          pltpu.VMEM((N_ATOMS,), f32),
          pltpu.VMEM((N_ATOMS,), f32),
          pltpu.VMEM((N_ATOMS,), f32),
          pltpu.VMEM((TPW,), i32),
          pltpu.VMEM((TPW,), i32),
          pltpu.VMEM((TPW,), i32),
          pltpu.VMEM((TPW,), i32),
          pltpu.VMEM((TPW,), i32),
          pltpu.VMEM((TPW,), f32),
          pltpu.VMEM((TPW,), f32),
          pltpu.VMEM((TPW,), f32),
          pltpu.VMEM((TPW,), f32),
          pltpu.SemaphoreType.DMA,
      ],
  )
  def k(px_hbm, py_hbm, pz_hbm, i_hbm, j_hbm, kj_hbm, ji_hbm, d2_hbm,
        dot_hbm, n1_hbm, n2_hbm, d2kj_hbm,
        px, py, pz, jiv, kjv, av, bv, kv, dkv, odot, on1, on2, sem):
    w = _wid()
    base = w * TPW
    pltpu.sync_copy(px_hbm, px)
    pltpu.sync_copy(py_hbm, py)
    pltpu.sync_copy(pz_hbm, pz)
    pltpu.sync_copy(ji_hbm.at[pl.ds(base, TPW)], jiv)
    pltpu.sync_copy(kj_hbm.at[pl.ds(base, TPW)], kjv)
    pltpu.async_copy(i_hbm.at[jiv], av, sem).wait()
    pltpu.async_copy(j_hbm.at[jiv], bv, sem).wait()
    pltpu.async_copy(j_hbm.at[kjv], kv, sem).wait()
    pltpu.async_copy(d2_hbm.at[kjv], dkv, sem).wait()

    def body(g, _):
      st = jnp.minimum(g * L, TPW - L)
      a16 = av[pl.ds(st, L)]
      b16 = bv[pl.ds(st, L)]
      k16 = kv[pl.ds(st, L)]
      bx = plsc.load_gather(px, [b16])
      by = plsc.load_gather(py, [b16])
      bz = plsc.load_gather(pz, [b16])
      v1x = plsc.load_gather(px, [a16]) - bx
      v1y = plsc.load_gather(py, [a16]) - by
      v1z = plsc.load_gather(pz, [a16]) - bz
      v2x = plsc.load_gather(px, [k16]) - bx
      v2y = plsc.load_gather(py, [k16]) - by
      v2z = plsc.load_gather(pz, [k16]) - bz
      odot[pl.ds(st, L)] = v1x * v2x + v1y * v2y + v1z * v2z
      on1[pl.ds(st, L)] = v1x * v1x + v1y * v1y + v1z * v1z
      on2[pl.ds(st, L)] = v2x * v2x + v2y * v2y + v2z * v2z
      return 0

    lax.fori_loop(0, (TPW + L - 1) // L, body, 0)
    pltpu.sync_copy(odot, dot_hbm.at[pl.ds(base, TPW)])
    pltpu.sync_copy(on1, n1_hbm.at[pl.ds(base, TPW)])
    pltpu.sync_copy(on2, n2_hbm.at[pl.ds(base, TPW)])
    pltpu.sync_copy(dkv, d2kj_hbm.at[pl.ds(base, TPW)])

  return k(pxa, pya, pza, i_arr, j_arr, idx_kj, idx_ji, d2)


_G_CHUNK = 200


def _sc_row_gather(tab, idx):
  n_chunks = TPW // _G_CHUNK

  @functools.partial(
      pl.kernel,
      out_type=jax.ShapeDtypeStruct((N_TRIPLETS, F), f32),
      mesh=_sc_mesh(),
      compiler_params=pltpu.CompilerParams(needs_layout_passes=False),
      scratch_types=[
          pltpu.VMEM((_G_CHUNK,), i32),
          pltpu.VMEM((_G_CHUNK,), i32),
          pltpu.VMEM((_G_CHUNK, F), f32),
          pltpu.VMEM((_G_CHUNK, F), f32),
          pltpu.SemaphoreType.DMA,
          pltpu.SemaphoreType.DMA,
          pltpu.SemaphoreType.DMA,
          pltpu.SemaphoreType.DMA,
      ],
  )
  def k(tab_hbm, idx_hbm, out_hbm, idx0, idx1, rows0, rows1,
        gs0, gs1, os0, os1):
    w = _wid()
    base = w * TPW
    ibufs = (idx0, idx1)
    rbufs = (rows0, rows1)
    gsems = (gs0, gs1)
    osems = (os0, os1)
    gd = [None] * n_chunks
    od = [None] * n_chunks

    def off(c):
      return pl.multiple_of(base + c * _G_CHUNK, 8)

    def start_gather(c):
      b = c % 2
      pltpu.sync_copy(idx_hbm.at[pl.ds(off(c), _G_CHUNK)], ibufs[b])
      gd[c] = pltpu.async_copy(tab_hbm.at[ibufs[b]], rbufs[b], gsems[b])

    start_gather(0)
    for c in range(n_chunks):
      b = c % 2
      if c + 1 < n_chunks:
        if c >= 1:
          od[c - 1].wait()
        start_gather(c + 1)
      gd[c].wait()
      od[c] = pltpu.async_copy(rbufs[b], out_hbm.at[pl.ds(off(c), _G_CHUNK)],
                               osems[b])
    od[n_chunks - 2].wait()
    od[n_chunks - 1].wait()

  return k(tab, idx)


_CH = 8000
_CHP = 8192
_EPC = N_EDGES // NC
_NCHUNK = _EPC // _CH
_ZROWS = _CHP // NS
_WROWS = 1000


def _sc_scatter_edges(rows_hbm_arr, idx):
  @functools.partial(
      pl.kernel,
      out_type=jax.ShapeDtypeStruct((N_EDGES, F), f32),
      mesh=_sc_mesh(),
      compiler_params=pltpu.CompilerParams(needs_layout_passes=False),
      scratch_types=[
          pltpu.VMEM((TPS,), i32),
          pltpu.VMEM((TPS + 9 * L,), i32),
          pltpu.VMEM((TPS + 9 * L,), i32),
          pltpu.VMEM((8 * L, F), f32),
          pltpu.VMEM((L, F), f32),
          pltpu.VMEM_SHARED((_CHP, F), f32),
          pltpu.SemaphoreType.DMA,
          pltpu.SemaphoreType.DMA,
      ],
  )
  def k(rows_hbm, idx_hbm, agg_hbm, idxv, tlist, llist, rbuf, zrow, acc,
        sem, sem2):
    c = lax.axis_index("c")
    s = lax.axis_index("s")
    sbase = s * TPS
    pltpu.sync_copy(idx_hbm.at[pl.ds(sbase, TPS)], idxv)

    def zfill(r, _):
      for gg in range(F // L):
        zrow[r, pl.ds(gg * L, L)] = jnp.zeros((L,), f32)
      return 0

    lax.fori_loop(0, L, zfill, 0)

    def chunk_body(ci, _):
      cbase = c * _EPC + ci * _CH
      def zcopy(r, _):
        pltpu.sync_copy(
            zrow, acc.at[pl.ds(pl.multiple_of(s * _ZROWS + r * L, 8), L)])
        return 0

      lax.fori_loop(0, _ZROWS // L, zcopy, 0)
      plsc.subcore_barrier()

      def scan_body(g, ptr):
        v = idxv[pl.ds(g * L, L)]
        local = v - cbase
        m = (local >= 0) & (local < _CH)
        tglob = lax.iota(i32, L) + (sbase + g * L)
        plsc.store_compressed(tlist.at[pl.ds(ptr, L)], tglob, mask=m)
        plsc.store_compressed(llist.at[pl.ds(ptr, L)], local, mask=m)
        return ptr + jnp.sum(jnp.where(m, 1, 0).astype(i32))

      ptr = lax.fori_loop(0, TPS // L, scan_body, jnp.int32(0))
      for kk in range(8):
        tlist[pl.ds(ptr + kk * L, L)] = jnp.zeros((L,), i32)
        llist[pl.ds(ptr + kk * L, L)] = jnp.full((L,), _CH, i32)

      nbo = (ptr + (8 * L - 1)) // (8 * L)

      def batch_body(m, _):
        gds = []
        for kk in range(8):
          tvec = tlist[pl.ds(m * (8 * L) + kk * L, L)]
          gds.append(pltpu.async_copy(
              rows_hbm.at[tvec], rbuf.at[pl.ds(kk * L, L)], sem))
        sds = []
        for kk in range(8):
          gds[kk].wait()
          lvec = llist[pl.ds(m * (8 * L) + kk * L, L)]
          sds.append(pltpu.async_copy(
              rbuf.at[pl.ds(kk * L, L)], acc.at[lvec], sem2, add=True))
        for d in sds:
          d.wait()
        return 0

      lax.fori_loop(0, nbo, batch_body, 0)
      plsc.subcore_barrier()
      @pl.when(s < 8)
      def _():
        pltpu.sync_copy(
            acc.at[pl.ds(pl.multiple_of(s * _WROWS, 8), _WROWS)],
            agg_hbm.at[pl.ds(pl.multiple_of(cbase + s * _WROWS, 8), _WROWS)])
      plsc.subcore_barrier()
      return 0

    lax.fori_loop(0, _NCHUNK, chunk_body, 0)

  return k(rows_hbm_arr, idx)


_NP_PAD = 10240
_AZROWS = _NP_PAD // NS


def _sc_scatter_atoms(rows_hbm_arr, i_arr):
  n_full = EPW // L
  tail_valid = EPW - n_full * L

  @functools.partial(
      pl.kernel,
      out_type=jax.ShapeDtypeStruct((NC, _NP_PAD, F), f32),
      mesh=_sc_mesh(),
      compiler_params=pltpu.CompilerParams(needs_layout_passes=False),
      scratch_types=[
          pltpu.VMEM((EPW,), i32),
          pltpu.VMEM((4 * L, F), f32),
          pltpu.VMEM((4 * L, F), f32),
          pltpu.VMEM((L, F), f32),
          pltpu.VMEM_SHARED((_NP_PAD, F), f32),
          pltpu.SemaphoreType.DMA,
          pltpu.SemaphoreType.DMA,
          pltpu.SemaphoreType.DMA,
      ],
  )
  def k(rows_hbm, i_hbm, part_hbm, iv, rbufa, rbufb, zrow, acc,
        sema, semb, sems):
    c = lax.axis_index("c")
    s = lax.axis_index("s")
    w = c * NS + s
    base = w * EPW
    pltpu.sync_copy(i_hbm.at[pl.ds(base, EPW)], iv)

    def zfill(r, _):
      for gg in range(F // L):
        zrow[r, pl.ds(gg * L, L)] = jnp.zeros((L,), f32)
      return 0

    lax.fori_loop(0, L, zfill, 0)

    def zcopy(r, _):
      pltpu.sync_copy(
          zrow, acc.at[pl.ds(pl.multiple_of(s * _AZROWS + r * L, 8), L)])
      return 0

    lax.fori_loop(0, _AZROWS // L, zcopy, 0)
    plsc.subcore_barrier()

    lane = lax.iota(i32, L)
    n_pair = n_full // 8

    def body(m, _):
      st_a = m * (8 * L)
      st_b = st_a + 4 * L
      lda = pltpu.async_copy(
          rows_hbm.at[pl.ds(pl.multiple_of(base + st_a, 8), 4 * L)],
          rbufa, sema)
      ldb = pltpu.async_copy(
          rows_hbm.at[pl.ds(pl.multiple_of(base + st_b, 8), 4 * L)],
          rbufb, semb)
      sds = []
      lda.wait()
      for kk in range(4):
        ivec = iv[pl.ds(st_a + kk * L, L)]
        sds.append(pltpu.async_copy(
            rbufa.at[pl.ds(kk * L, L)], acc.at[ivec], sems, add=True))
      ldb.wait()
      for kk in range(4):
        ivec = iv[pl.ds(st_b + kk * L, L)]
        sds.append(pltpu.async_copy(
            rbufb.at[pl.ds(kk * L, L)], acc.at[ivec], sems, add=True))
      for d in sds:
        d.wait()
      return 0

    lax.fori_loop(0, n_pair, body, 0)
    for g in range(n_pair * 8, n_full):
      st = g * L
      pltpu.sync_copy(
          rows_hbm.at[pl.ds(pl.multiple_of(base + st, 8), L)],
          rbufa.at[pl.ds(0, L)])
      ivec = iv[pl.ds(st, L)]
      pltpu.sync_copy(rbufa.at[pl.ds(0, L)], acc.at[ivec], add=True)
    if tail_valid:
      st = EPW - L
      pltpu.sync_copy(rows_hbm.at[pl.ds(pl.multiple_of(base + st, 8), L)],
                      rbufa.at[pl.ds(0, L)])
      ivec = iv[pl.ds(st, L)]
      ivec = jnp.where(lane < (L - tail_valid), jnp.full((L,), N_ATOMS, i32),
                       ivec)
      pltpu.sync_copy(rbufa.at[pl.ds(0, L)], acc.at[ivec], add=True)

    plsc.subcore_barrier()
    pltpu.sync_copy(
        acc.at[pl.ds(pl.multiple_of(s * _AZROWS, 8), _AZROWS)],
        part_hbm.at[c, pl.ds(pl.multiple_of(s * _AZROWS, 8), _AZROWS)])

  return k(rows_hbm_arr, i_arr)


_BE = 256
_NEB = N_EDGES // _BE

def _freq_row():
  return (lax.broadcasted_iota(jnp.int32, (1, R), 1).astype(f32) + 1.0) * np.pi


def _sbf_consts():
  q = lax.broadcasted_iota(jnp.int32, (1, S * R), 1)
  r = (q % R).astype(f32)
  s = (q // R).astype(f32)
  bwf = (r + 1.0) * np.pi + s * (S - 1) * np.pi
  return bwf, s


def _vec_spec(b):
  return pl.BlockSpec((b,), lambda k: (k,))


def _mat_spec(b, d):
  return pl.BlockSpec((b, d), lambda k: (k, 0))


def _full_spec(shape):
  nd = len(shape)
  return pl.BlockSpec(shape, lambda k: (0,) * nd)


def _tc_edge(d2, cvec, emb_w, emb_b, rbf_w0):
  def body(d2_ref, c_ref, ew_ref, eb_ref, rw0_ref, rbf_ref, m0_ref, t0_ref):
    d2t = d2_ref[...]
    dist = jnp.sqrt(d2t)
    ds_ = dist / CUTOFF
    env = _envelope(ds_)
    rbf = env[:, None] * jnp.sin(_freq_row() * ds_[:, None]) / dist[:, None]
    rbf_ref[...] = rbf
    m0 = c_ref[...][None, :] * _swish(
        jnp.dot(rbf, ew_ref[...], preferred_element_type=f32)
        + eb_ref[...][None, :])
    m0_ref[...] = m0
    t0_ref[...] = m0 * jnp.dot(rbf, rw0_ref[...], preferred_element_type=f32)

  return pl.pallas_call(
      body,
      grid=(_NEB,),
      in_specs=[_vec_spec(_BE), _full_spec((F,)), _full_spec((R, F)),
                _full_spec((F,)), _full_spec((R, F))],
      out_specs=[_mat_spec(_BE, R), _mat_spec(_BE, F), _mat_spec(_BE, F)],
      out_shape=[jax.ShapeDtypeStruct((N_EDGES, R), f32),
                 jax.ShapeDtypeStruct((N_EDGES, F), f32),
                 jax.ShapeDtypeStruct((N_EDGES, F), f32)],
  )(d2, cvec, emb_w, emb_b, rbf_w0)


def _tc_triplet(dot, n1, n2, d2kj, sw0, sb0, sw1, sb1):
  def body(dot_ref, n1_ref, n2_ref, dk_ref, w0_ref, b0_ref, w1_ref, b1_ref,
           sf0_ref, sf1_ref):
    dott = dot_ref[...]
    nrm = jnp.sqrt(n1_ref[...] + 1e-12) * jnp.sqrt(n2_ref[...] + 1e-12)
    ca = jnp.clip(dott / (nrm + 1e-7), -1.0 + 1e-6, 1.0 - 1e-6)
    dk = jnp.sqrt(dk_ref[...]) / CUTOFF
    env = _envelope(dk)
    bwf, shwf = _sbf_consts()
    cprev = jnp.ones_like(ca)
    ccur = ca
    sph = (shwf == 0.0).astype(f32) * cprev[:, None]
    sph = sph + (shwf == 1.0).astype(f32) * ccur[:, None]
    for s_ in range(2, S):
      cnext = 2.0 * ca * ccur - cprev
      cprev, ccur = ccur, cnext
      sph = sph + (shwf == float(s_)).astype(f32) * ccur[:, None]
    sbf = env[:, None] * jnp.sin(bwf * dk[:, None]) * sph
    sf0_ref[...] = _swish(
        jnp.dot(sbf, w0_ref[...], preferred_element_type=f32)
        + b0_ref[...][None, :])
    sf1_ref[...] = _swish(
        jnp.dot(sbf, w1_ref[...], preferred_element_type=f32)
        + b1_ref[...][None, :])

  return pl.pallas_call(
      body,
      grid=(_NEB,),
      in_specs=[_vec_spec(_BE)] * 4
      + [_full_spec((S * R, 8)), _full_spec((8,)),
         _full_spec((S * R, 8)), _full_spec((8,))],
      out_specs=[_mat_spec(_BE, 8), _mat_spec(_BE, 8)],
      out_shape=[jax.ShapeDtypeStruct((N_TRIPLETS, 8), f32),
                 jax.ShapeDtypeStruct((N_TRIPLETS, 8), f32)],
  )(dot, n1, n2, d2kj, sw0, sb0, sw1, sb1)


def _tc_mkj(m, mw, mb):
  def body(m_ref, w_ref, b_ref, o_ref):
    o_ref[...] = _swish(
        jnp.dot(m_ref[...], w_ref[...], preferred_element_type=f32)
        + b_ref[...][None, :])

  return pl.pallas_call(
      body,
      grid=(_NEB,),
      in_specs=[_mat_spec(_BE, F), _full_spec((F, F)), _full_spec((F,))],
      out_specs=_mat_spec(_BE, F),
      out_shape=jax.ShapeDtypeStruct((N_EDGES, F), f32),
  )(m, mw, mb)


def _tc_bil(x2, sf, bmat, bil_b):
  def body(x_ref, sf_ref, bm_ref, bb_ref, o_ref):
    y = jnp.dot(x_ref[...], bm_ref[...], preferred_element_type=f32)
    sft = sf_ref[...]
    acc = bb_ref[...][None, :]
    for ii in range(8):
      acc = acc + sft[:, ii][:, None] * y[:, ii * F:(ii + 1) * F]
    o_ref[...] = acc

  return pl.pallas_call(
      body,
      grid=(_NEB,),
      in_specs=[_mat_spec(_BE, F), _mat_spec(_BE, 8),
                _full_spec((F, 8 * F)), _full_spec((F,))],
      out_specs=_mat_spec(_BE, F),
      out_shape=jax.ShapeDtypeStruct((N_TRIPLETS, F), f32),
  )(x2, sf, bmat, bil_b)


def _tc_update(agg, m, rbf, dw, db, res, rbf_wn):
  rp = [res[0]['w1'], res[0]['b1'], res[0]['w2'], res[0]['b2'],
        res[1]['w1'], res[1]['b1'], res[1]['w2'], res[1]['b2']]

  def body(agg_ref, m_ref, rbf_ref, dw_ref, db_ref,
           r0w1, r0b1, r0w2, r0b2, r1w1, r1b1, r1w2, r1b2, rwn_ref,
           mn_ref, tn_ref):
    mm = m_ref[...] + _swish(
        jnp.dot(agg_ref[...], dw_ref[...], preferred_element_type=f32)
        + db_ref[...][None, :])
    for (w1, b1, w2, b2) in ((r0w1, r0b1, r0w2, r0b2),
                             (r1w1, r1b1, r1w2, r1b2)):
      h = _swish(jnp.dot(mm, w1[...], preferred_element_type=f32)
                 + b1[...][None, :])
      mm = mm + jnp.dot(h, w2[...], preferred_element_type=f32) + b2[...][None, :]
    mn_ref[...] = mm
    tn_ref[...] = mm * jnp.dot(rbf_ref[...], rwn_ref[...],
                               preferred_element_type=f32)

  return pl.pallas_call(
      body,
      grid=(_NEB,),
      in_specs=[_mat_spec(_BE, F), _mat_spec(_BE, F), _mat_spec(_BE, R),
                _full_spec((F, F)), _full_spec((F,)),
                _full_spec((F, F)), _full_spec((F,)),
                _full_spec((F, F)), _full_spec((F,)),
                _full_spec((F, F)), _full_spec((F,)),
                _full_spec((F, F)), _full_spec((F,)),
                _full_spec((R, F))],
      out_specs=[_mat_spec(_BE, F), _mat_spec(_BE, F)],
      out_shape=[jax.ShapeDtypeStruct((N_EDGES, F), f32),
                 jax.ShapeDtypeStruct((N_EDGES, F), f32)],
  )(agg, m, rbf, dw, db, *rp, rbf_wn)


_BA = 1000
_NAB = N_ATOMS // _BA


def _tc_atom_out(parts, outp):
  ws = []
  for pb in outp:
    ws += [pb['w1'], pb['b1'], pb['w2'], pb['b2']]

  def body(p0_ref, p1_ref, p2_ref,
           w10, b10, w20, b20, w11, b11, w21, b21, w12, b12, w22, b22,
           o_ref):
    @pl.when(pl.program_id(0) == 0)
    def _():
      o_ref[...] = jnp.zeros((1, 1), f32)

    tot = jnp.zeros((), f32)
    for (p_ref, w1, b1, w2, b2) in (
        (p0_ref, w10, b10, w20, b20),
        (p1_ref, w11, b11, w21, b21),
        (p2_ref, w12, b12, w22, b22)):
      af = p_ref[0] + p_ref[1]
      h = _swish(af)
      h2 = _swish(jnp.dot(h, w1[...], preferred_element_type=f32)
                  + b1[...][None, :])
      o = jnp.dot(h2, w2[...], preferred_element_type=f32) + b2[...][None, :]
      tot = tot + jnp.sum(o)
    o_ref[...] = o_ref[...] + tot.reshape(1, 1)

  pspec = pl.BlockSpec((NC, _BA, F), lambda k: (0, k, 0))
  return pl.pallas_call(
      body,
      grid=(_NAB,),
      in_specs=[pspec, pspec, pspec,
                _full_spec((F, F // 2)), _full_spec((F // 2,)),
                _full_spec((F // 2, 1)), _full_spec((1,)),
                _full_spec((F, F // 2)), _full_spec((F // 2,)),
                _full_spec((F // 2, 1)), _full_spec((1,)),
                _full_spec((F, F // 2)), _full_spec((F // 2,)),
                _full_spec((F // 2, 1)), _full_spec((1,))],
      out_specs=_full_spec((1, 1)),
      out_shape=jax.ShapeDtypeStruct((1, 1), f32),
  )(*parts, *ws)


def kernel(z, pos, i, j, idx_kj, idx_ji, params):
  del z
  pxa, pya, pza = pos[:, 0], pos[:, 1], pos[:, 2]

  d2 = _sc_edge_d2(pxa, pya, pza, i, j)
  dot, n1, n2, d2kj = _sc_triplet_geom(pxa, pya, pza, i, j, idx_kj, idx_ji, d2)

  emb = params['emb']
  cvec = (emb['atom'][0] * emb['atom'][0]).astype(f32)
  outp = params['out']
  rbf, m, t0 = _tc_edge(d2, cvec, emb['rbf_w'], emb['rbf_b'], outp[0]['rbf_w'])

  ib0, ib1 = params['inter'][0], params['inter'][1]
  sf0, sf1 = _tc_triplet(dot, n1, n2, d2kj,
                         ib0['sbf_w'], ib0['sbf_b'], ib1['sbf_w'], ib1['sbf_b'])

  parts = [_sc_scatter_atoms(t0, i)]
  sfs = (sf0, sf1)
  for blk, ib in enumerate((ib0, ib1)):
    mkj = _tc_mkj(m, ib['m_w'], ib['m_b'])
    x2 = _sc_row_gather(mkj, idx_kj)
    bmat = jnp.transpose(ib['bil_w'], (2, 1, 0)).reshape(F, 8 * F)
    bil = _tc_bil(x2, sfs[blk], bmat, ib['bil_b'])
    agg = _sc_scatter_edges(bil, idx_ji)
    m, tn = _tc_update(agg, m, rbf, ib['down_w'], ib['down_b'], ib['res'],
                       outp[blk + 1]['rbf_w'])
    parts.append(_sc_scatter_atoms(tn, i))

  total = _tc_atom_out(parts, outp)
  return total[0, 0]


# speedup vs baseline: 1.0913x; 1.0913x over previous
import functools

import jax
import jax.numpy as jnp
import numpy as np
from jax import lax
from jax.experimental import pallas as pl
from jax.experimental.pallas import tpu as pltpu
from jax.experimental.pallas import tpu_sc as plsc

N_ATOMS = 10000
N_EDGES = 160000
N_TRIPLETS = 160000
F = 128
R = 6
S = 7
CUTOFF = 5.0
P = 7

NC, NS, L = 2, 16, 16
NW = NC * NS
EPW = N_EDGES // NW
TPW = N_TRIPLETS // NW
TPS = N_TRIPLETS // NS

def _sc_mesh():
  return plsc.VectorSubcoreMesh(
      core_axis_name="c", subcore_axis_name="s", num_cores=NC, num_subcores=NS)

f32 = jnp.float32
i32 = jnp.int32


def _swish(x):
  return x * jax.nn.sigmoid(x)


def _envelope(x):
  p = P
  a = -(p + 1) * (p + 2) / 2.0
  b = p * (p + 2)
  c = -p * (p + 1) / 2.0
  xp0 = x ** (p - 1)
  xp1 = xp0 * x
  xp2 = xp1 * x
  return 1.0 / x + a * xp0 + b * xp1 + c * xp2


def _wid():
  return lax.axis_index("c") * NS + lax.axis_index("s")


def _sc_edge_d2(pxa, pya, pza, i_arr, j_arr):
  @functools.partial(
      pl.kernel,
      out_type=jax.ShapeDtypeStruct((N_EDGES,), f32),
      mesh=_sc_mesh(),
      compiler_params=pltpu.CompilerParams(needs_layout_passes=False),
      scratch_types=[
          pltpu.VMEM((N_ATOMS,), f32),
          pltpu.VMEM((N_ATOMS,), f32),
          pltpu.VMEM((N_ATOMS,), f32),
          pltpu.VMEM((EPW,), i32),
          pltpu.VMEM((EPW,), i32),
          pltpu.VMEM((EPW,), f32),
      ],
  )
  def k(px_hbm, py_hbm, pz_hbm, i_hbm, j_hbm, d2_hbm, px, py, pz, iv, jv, ov):
    w = _wid()
    base = w * EPW
    pltpu.sync_copy(px_hbm, px)
    pltpu.sync_copy(py_hbm, py)
    pltpu.sync_copy(pz_hbm, pz)
    pltpu.sync_copy(i_hbm.at[pl.ds(base, EPW)], iv)
    pltpu.sync_copy(j_hbm.at[pl.ds(base, EPW)], jv)

    def body(g, _):
      st = jnp.minimum(g * L, EPW - L)
      a16 = iv[pl.ds(st, L)]
      b16 = jv[pl.ds(st, L)]
      dx = plsc.load_gather(px, [a16]) - plsc.load_gather(px, [b16])
      dy = plsc.load_gather(py, [a16]) - plsc.load_gather(py, [b16])
      dz = plsc.load_gather(pz, [a16]) - plsc.load_gather(pz, [b16])
      ov[pl.ds(st, L)] = dx * dx + dy * dy + dz * dz
      return 0

    lax.fori_loop(0, (EPW + L - 1) // L, body, 0)
    pltpu.sync_copy(ov, d2_hbm.at[pl.ds(base, EPW)])

  return k(pxa, pya, pza, i_arr, j_arr)


def _sc_triplet_geom(pxa, pya, pza, i_arr, j_arr, idx_kj, idx_ji, d2):
  out_t = tuple(jax.ShapeDtypeStruct((N_TRIPLETS,), f32) for _ in range(4))

  @functools.partial(
      pl.kernel,
      out_type=out_t,
      mesh=_sc_mesh(),
      compiler_params=pltpu.CompilerParams(needs_layout_passes=False),
      scratch_types=[
          pltpu.VMEM((N_ATOMS,), f32),
          pltpu.VMEM((N_ATOMS,), f32),
          pltpu.VMEM((N_ATOMS,), f32),
          pltpu.VMEM((TPW,), i32),
          pltpu.VMEM((TPW,), i32),
          pltpu.VMEM((TPW,), i32),
          pltpu.VMEM((TPW,), i32),
          pltpu.VMEM((TPW,), i32),
          pltpu.VMEM((TPW,), f32),
          pltpu.VMEM((TPW,), f32),
          pltpu.VMEM((TPW,), f32),
          pltpu.VMEM((TPW,), f32),
          pltpu.SemaphoreType.DMA,
      ],
  )
  def k(px_hbm, py_hbm, pz_hbm, i_hbm, j_hbm, kj_hbm, ji_hbm, d2_hbm,
        dot_hbm, n1_hbm, n2_hbm, d2kj_hbm,
        px, py, pz, jiv, kjv, av, bv, kv, dkv, odot, on1, on2, sem):
    w = _wid()
    base = w * TPW
    pltpu.sync_copy(px_hbm, px)
    pltpu.sync_copy(py_hbm, py)
    pltpu.sync_copy(pz_hbm, pz)
    pltpu.sync_copy(ji_hbm.at[pl.ds(base, TPW)], jiv)
    pltpu.sync_copy(kj_hbm.at[pl.ds(base, TPW)], kjv)
    pltpu.async_copy(i_hbm.at[jiv], av, sem).wait()
    pltpu.async_copy(j_hbm.at[jiv], bv, sem).wait()
    pltpu.async_copy(j_hbm.at[kjv], kv, sem).wait()
    pltpu.async_copy(d2_hbm.at[kjv], dkv, sem).wait()

    def body(g, _):
      st = jnp.minimum(g * L, TPW - L)
      a16 = av[pl.ds(st, L)]
      b16 = bv[pl.ds(st, L)]
      k16 = kv[pl.ds(st, L)]
      bx = plsc.load_gather(px, [b16])
      by = plsc.load_gather(py, [b16])
      bz = plsc.load_gather(pz, [b16])
      v1x = plsc.load_gather(px, [a16]) - bx
      v1y = plsc.load_gather(py, [a16]) - by
      v1z = plsc.load_gather(pz, [a16]) - bz
      v2x = plsc.load_gather(px, [k16]) - bx
      v2y = plsc.load_gather(py, [k16]) - by
      v2z = plsc.load_gather(pz, [k16]) - bz
      odot[pl.ds(st, L)] = v1x * v2x + v1y * v2y + v1z * v2z
      on1[pl.ds(st, L)] = v1x * v1x + v1y * v1y + v1z * v1z
      on2[pl.ds(st, L)] = v2x * v2x + v2y * v2y + v2z * v2z
      return 0

    lax.fori_loop(0, (TPW + L - 1) // L, body, 0)
    pltpu.sync_copy(odot, dot_hbm.at[pl.ds(base, TPW)])
    pltpu.sync_copy(on1, n1_hbm.at[pl.ds(base, TPW)])
    pltpu.sync_copy(on2, n2_hbm.at[pl.ds(base, TPW)])
    pltpu.sync_copy(dkv, d2kj_hbm.at[pl.ds(base, TPW)])

  return k(pxa, pya, pza, i_arr, j_arr, idx_kj, idx_ji, d2)


_G_CHUNK = 200


def _sc_row_gather(tab, idx):
  n_chunks = TPW // _G_CHUNK

  @functools.partial(
      pl.kernel,
      out_type=jax.ShapeDtypeStruct((N_TRIPLETS, F), f32),
      mesh=_sc_mesh(),
      compiler_params=pltpu.CompilerParams(needs_layout_passes=False),
      scratch_types=[
          pltpu.VMEM((2 * _G_CHUNK,), i32),
          pltpu.VMEM((2 * _G_CHUNK, F), f32),
          pltpu.SemaphoreType.DMA,
          pltpu.SemaphoreType.DMA,
      ],
  )
  def k(tab_hbm, idx_hbm, out_hbm, idx2, rows2, gsem, osem):
    w = _wid()
    base = w * TPW

    def body(m, _):
      boff = pl.multiple_of((m % 2) * _G_CHUNK, 8)
      ib = idx2.at[pl.ds(boff, _G_CHUNK)]
      rb = rows2.at[pl.ds(boff, _G_CHUNK)]

      @pl.when(m >= 2)
      def _():
        pltpu.make_async_copy(tab_hbm.at[pl.ds(0, _G_CHUNK)], rb, osem).wait()

      off = pl.multiple_of(base + m * _G_CHUNK, 8)
      pltpu.sync_copy(idx_hbm.at[pl.ds(off, _G_CHUNK)], ib)
      pltpu.async_copy(tab_hbm.at[ib], rb, gsem).wait()
      pltpu.async_copy(rb, out_hbm.at[pl.ds(off, _G_CHUNK)], osem)
      return 0

    lax.fori_loop(0, n_chunks, body, 0)
    pltpu.make_async_copy(tab_hbm.at[pl.ds(0, _G_CHUNK)],
                          rows2.at[pl.ds(0, _G_CHUNK)], osem).wait()
    pltpu.make_async_copy(tab_hbm.at[pl.ds(0, _G_CHUNK)],
                          rows2.at[pl.ds(_G_CHUNK, _G_CHUNK)], osem).wait()

  return k(tab, idx)


_CH = 8000
_CHP = 8192
_EPC = N_EDGES // NC
_NCHUNK = _EPC // _CH
_ZROWS = _CHP // NS
_WROWS = 1000


def _sc_scatter_edges(rows_hbm_arr, idx):
  @functools.partial(
      pl.kernel,
      out_type=jax.ShapeDtypeStruct((N_EDGES, F), f32),
      mesh=_sc_mesh(),
      compiler_params=pltpu.CompilerParams(needs_layout_passes=False),
      scratch_types=[
          pltpu.VMEM((TPS,), i32),
          pltpu.VMEM((TPS + 9 * L,), i32),
          pltpu.VMEM((TPS + 9 * L,), i32),
          pltpu.VMEM((4 * L, F), f32),
          pltpu.VMEM((L, F), f32),
          pltpu.VMEM_SHARED((_CHP, F), f32),
          pltpu.SemaphoreType.DMA,
          pltpu.SemaphoreType.DMA,
      ],
  )
  def k(rows_hbm, idx_hbm, agg_hbm, idxv, tlist, llist, rbuf, zrow, acc,
        sem, sem2):
    c = lax.axis_index("c")
    s = lax.axis_index("s")
    sbase = s * TPS
    pltpu.sync_copy(idx_hbm.at[pl.ds(sbase, TPS)], idxv)

    def zfill(r, _):
      for gg in range(F // L):
        zrow[r, pl.ds(gg * L, L)] = jnp.zeros((L,), f32)
      return 0

    lax.fori_loop(0, L, zfill, 0)

    def chunk_body(ci, _):
      cbase = c * _EPC + ci * _CH
      def zcopy(r, _):
        pltpu.sync_copy(
            zrow, acc.at[pl.ds(pl.multiple_of(s * _ZROWS + r * L, 8), L)])
        return 0

      lax.fori_loop(0, _ZROWS // L, zcopy, 0)
      plsc.subcore_barrier()

      def scan_body(g, ptr):
        v = idxv[pl.ds(g * L, L)]
        local = v - cbase
        m = (local >= 0) & (local < _CH)
        tglob = lax.iota(i32, L) + (sbase + g * L)
        plsc.store_compressed(tlist.at[pl.ds(ptr, L)], tglob, mask=m)
        plsc.store_compressed(llist.at[pl.ds(ptr, L)], local, mask=m)
        return ptr + jnp.sum(jnp.where(m, 1, 0).astype(i32))

      ptr = lax.fori_loop(0, TPS // L, scan_body, jnp.int32(0))
      for kk in range(8):
        tlist[pl.ds(ptr + kk * L, L)] = jnp.zeros((L,), i32)
        llist[pl.ds(ptr + kk * L, L)] = jnp.full((L,), _CH, i32)

      nbo = (ptr + (4 * L - 1)) // (4 * L)

      def batch_body(m, _):
        tref = tlist.at[pl.ds(m * (4 * L), 4 * L)]
        pltpu.async_copy(rows_hbm.at[tref], rbuf, sem).wait()
        sds = []
        for kk in range(4):
          lvec = llist[pl.ds(m * (4 * L) + kk * L, L)]
          sds.append(pltpu.async_copy(
              rbuf.at[pl.ds(kk * L, L)], acc.at[lvec], sem2, add=True))
        for d in sds:
          d.wait()
        return 0

      lax.fori_loop(0, nbo, batch_body, 0)
      plsc.subcore_barrier()
      @pl.when(s < 8)
      def _():
        pltpu.sync_copy(
            acc.at[pl.ds(pl.multiple_of(s * _WROWS, 8), _WROWS)],
            agg_hbm.at[pl.ds(pl.multiple_of(cbase + s * _WROWS, 8), _WROWS)])
      plsc.subcore_barrier()
      return 0

    lax.fori_loop(0, _NCHUNK, chunk_body, 0)

  return k(rows_hbm_arr, idx)


_NP_PAD = 10240
_AZROWS = _NP_PAD // NS


def _sc_scatter_atoms(rows_hbm_arr, i_arr):
  n_full = EPW // L
  tail_valid = EPW - n_full * L

  @functools.partial(
      pl.kernel,
      out_type=jax.ShapeDtypeStruct((NC, _NP_PAD, F), f32),
      mesh=_sc_mesh(),
      compiler_params=pltpu.CompilerParams(needs_layout_passes=False),
      scratch_types=[
          pltpu.VMEM((EPW,), i32),
          pltpu.VMEM((4 * L, F), f32),
          pltpu.VMEM((4 * L, F), f32),
          pltpu.VMEM((L, F), f32),
          pltpu.VMEM_SHARED((_NP_PAD, F), f32),
          pltpu.SemaphoreType.DMA,
          pltpu.SemaphoreType.DMA,
          pltpu.SemaphoreType.DMA,
      ],
  )
  def k(rows_hbm, i_hbm, part_hbm, iv, rbufa, rbufb, zrow, acc,
        sema, semb, sems):
    c = lax.axis_index("c")
    s = lax.axis_index("s")
    w = c * NS + s
    base = w * EPW
    pltpu.sync_copy(i_hbm.at[pl.ds(base, EPW)], iv)

    def zfill(r, _):
      for gg in range(F // L):
        zrow[r, pl.ds(gg * L, L)] = jnp.zeros((L,), f32)
      return 0

    lax.fori_loop(0, L, zfill, 0)

    def zcopy(r, _):
      pltpu.sync_copy(
          zrow, acc.at[pl.ds(pl.multiple_of(s * _AZROWS + r * L, 8), L)])
      return 0

    lax.fori_loop(0, _AZROWS // L, zcopy, 0)
    plsc.subcore_barrier()

    lane = lax.iota(i32, L)
    n_pair = n_full // 8

    def body(m, _):
      st_a = m * (8 * L)
      st_b = st_a + 4 * L
      lda = pltpu.async_copy(
          rows_hbm.at[pl.ds(pl.multiple_of(base + st_a, 8), 4 * L)],
          rbufa, sema)
      ldb = pltpu.async_copy(
          rows_hbm.at[pl.ds(pl.multiple_of(base + st_b, 8), 4 * L)],
          rbufb, semb)
      sds = []
      lda.wait()
      for kk in range(4):
        ivec = iv[pl.ds(st_a + kk * L, L)]
        sds.append(pltpu.async_copy(
            rbufa.at[pl.ds(kk * L, L)], acc.at[ivec], sems, add=True))
      ldb.wait()
      for kk in range(4):
        ivec = iv[pl.ds(st_b + kk * L, L)]
        sds.append(pltpu.async_copy(
            rbufb.at[pl.ds(kk * L, L)], acc.at[ivec], sems, add=True))
      for d in sds:
        d.wait()
      return 0

    lax.fori_loop(0, n_pair, body, 0)
    for g in range(n_pair * 8, n_full):
      st = g * L
      pltpu.sync_copy(
          rows_hbm.at[pl.ds(pl.multiple_of(base + st, 8), L)],
          rbufa.at[pl.ds(0, L)])
      ivec = iv[pl.ds(st, L)]
      pltpu.sync_copy(rbufa.at[pl.ds(0, L)], acc.at[ivec], add=True)
    if tail_valid:
      st = EPW - L
      pltpu.sync_copy(rows_hbm.at[pl.ds(pl.multiple_of(base + st, 8), L)],
                      rbufa.at[pl.ds(0, L)])
      ivec = iv[pl.ds(st, L)]
      ivec = jnp.where(lane < (L - tail_valid), jnp.full((L,), N_ATOMS, i32),
                       ivec)
      pltpu.sync_copy(rbufa.at[pl.ds(0, L)], acc.at[ivec], add=True)

    plsc.subcore_barrier()
    pltpu.sync_copy(
        acc.at[pl.ds(pl.multiple_of(s * _AZROWS, 8), _AZROWS)],
        part_hbm.at[c, pl.ds(pl.multiple_of(s * _AZROWS, 8), _AZROWS)])

  return k(rows_hbm_arr, i_arr)


_BE = 256
_NEB = N_EDGES // _BE

def _freq_row():
  return (lax.broadcasted_iota(jnp.int32, (1, R), 1).astype(f32) + 1.0) * np.pi


def _sbf_consts():
  q = lax.broadcasted_iota(jnp.int32, (1, S * R), 1)
  r = (q % R).astype(f32)
  s = (q // R).astype(f32)
  bwf = (r + 1.0) * np.pi + s * (S - 1) * np.pi
  return bwf, s


def _vec_spec(b):
  return pl.BlockSpec((b,), lambda k: (k,))


def _mat_spec(b, d):
  return pl.BlockSpec((b, d), lambda k: (k, 0))


def _full_spec(shape):
  nd = len(shape)
  return pl.BlockSpec(shape, lambda k: (0,) * nd)


def _tc_edge(d2, cvec, emb_w, emb_b, rbf_w0):
  def body(d2_ref, c_ref, ew_ref, eb_ref, rw0_ref, rbf_ref, m0_ref, t0_ref):
    d2t = d2_ref[...]
    dist = jnp.sqrt(d2t)
    ds_ = dist / CUTOFF
    env = _envelope(ds_)
    rbf = env[:, None] * jnp.sin(_freq_row() * ds_[:, None]) / dist[:, None]
    rbf_ref[...] = rbf
    m0 = c_ref[...][None, :] * _swish(
        jnp.dot(rbf, ew_ref[...], preferred_element_type=f32)
        + eb_ref[...][None, :])
    m0_ref[...] = m0
    t0_ref[...] = m0 * jnp.dot(rbf, rw0_ref[...], preferred_element_type=f32)

  return pl.pallas_call(
      body,
      grid=(_NEB,),
      in_specs=[_vec_spec(_BE), _full_spec((F,)), _full_spec((R, F)),
                _full_spec((F,)), _full_spec((R, F))],
      out_specs=[_mat_spec(_BE, R), _mat_spec(_BE, F), _mat_spec(_BE, F)],
      out_shape=[jax.ShapeDtypeStruct((N_EDGES, R), f32),
                 jax.ShapeDtypeStruct((N_EDGES, F), f32),
                 jax.ShapeDtypeStruct((N_EDGES, F), f32)],
  )(d2, cvec, emb_w, emb_b, rbf_w0)


def _tc_triplet(dot, n1, n2, d2kj, sw0, sb0, sw1, sb1):
  def body(dot_ref, n1_ref, n2_ref, dk_ref, w0_ref, b0_ref, w1_ref, b1_ref,
           sf0_ref, sf1_ref):
    dott = dot_ref[...]
    nrm = jnp.sqrt(n1_ref[...] + 1e-12) * jnp.sqrt(n2_ref[...] + 1e-12)
    ca = jnp.clip(dott / (nrm + 1e-7), -1.0 + 1e-6, 1.0 - 1e-6)
    dk = jnp.sqrt(dk_ref[...]) / CUTOFF
    env = _envelope(dk)
    bwf, shwf = _sbf_consts()
    cprev = jnp.ones_like(ca)
    ccur = ca
    sph = (shwf == 0.0).astype(f32) * cprev[:, None]
    sph = sph + (shwf == 1.0).astype(f32) * ccur[:, None]
    for s_ in range(2, S):
      cnext = 2.0 * ca * ccur - cprev
      cprev, ccur = ccur, cnext
      sph = sph + (shwf == float(s_)).astype(f32) * ccur[:, None]
    sbf = env[:, None] * jnp.sin(bwf * dk[:, None]) * sph
    sf0_ref[...] = _swish(
        jnp.dot(sbf, w0_ref[...], preferred_element_type=f32)
        + b0_ref[...][None, :])
    sf1_ref[...] = _swish(
        jnp.dot(sbf, w1_ref[...], preferred_element_type=f32)
        + b1_ref[...][None, :])

  return pl.pallas_call(
      body,
      grid=(_NEB,),
      in_specs=[_vec_spec(_BE)] * 4
      + [_full_spec((S * R, 8)), _full_spec((8,)),
         _full_spec((S * R, 8)), _full_spec((8,))],
      out_specs=[_mat_spec(_BE, 8), _mat_spec(_BE, 8)],
      out_shape=[jax.ShapeDtypeStruct((N_TRIPLETS, 8), f32),
                 jax.ShapeDtypeStruct((N_TRIPLETS, 8), f32)],
  )(dot, n1, n2, d2kj, sw0, sb0, sw1, sb1)


def _tc_mkj(m, mw, mb):
  def body(m_ref, w_ref, b_ref, o_ref):
    o_ref[...] = _swish(
        jnp.dot(m_ref[...], w_ref[...], preferred_element_type=f32)
        + b_ref[...][None, :])

  return pl.pallas_call(
      body,
      grid=(_NEB,),
      in_specs=[_mat_spec(_BE, F), _full_spec((F, F)), _full_spec((F,))],
      out_specs=_mat_spec(_BE, F),
      out_shape=jax.ShapeDtypeStruct((N_EDGES, F), f32),
  )(m, mw, mb)


def _tc_bil(x2, sf, bmat, bil_b):
  def body(x_ref, sf_ref, bm_ref, bb_ref, o_ref):
    y = jnp.dot(x_ref[...], bm_ref[...], preferred_element_type=f32)
    sft = sf_ref[...]
    acc = bb_ref[...][None, :]
    for ii in range(8):
      acc = acc + sft[:, ii][:, None] * y[:, ii * F:(ii + 1) * F]
    o_ref[...] = acc

  return pl.pallas_call(
      body,
      grid=(_NEB,),
      in_specs=[_mat_spec(_BE, F), _mat_spec(_BE, 8),
                _full_spec((F, 8 * F)), _full_spec((F,))],
      out_specs=_mat_spec(_BE, F),
      out_shape=jax.ShapeDtypeStruct((N_TRIPLETS, F), f32),
  )(x2, sf, bmat, bil_b)


def _tc_update(agg, m, rbf, dw, db, res, rbf_wn):
  rp = [res[0]['w1'], res[0]['b1'], res[0]['w2'], res[0]['b2'],
        res[1]['w1'], res[1]['b1'], res[1]['w2'], res[1]['b2']]

  def body(agg_ref, m_ref, rbf_ref, dw_ref, db_ref,
           r0w1, r0b1, r0w2, r0b2, r1w1, r1b1, r1w2, r1b2, rwn_ref,
           mn_ref, tn_ref):
    mm = m_ref[...] + _swish(
        jnp.dot(agg_ref[...], dw_ref[...], preferred_element_type=f32)
        + db_ref[...][None, :])
    for (w1, b1, w2, b2) in ((r0w1, r0b1, r0w2, r0b2),
                             (r1w1, r1b1, r1w2, r1b2)):
      h = _swish(jnp.dot(mm, w1[...], preferred_element_type=f32)
                 + b1[...][None, :])
      mm = mm + jnp.dot(h, w2[...], preferred_element_type=f32) + b2[...][None, :]
    mn_ref[...] = mm
    tn_ref[...] = mm * jnp.dot(rbf_ref[...], rwn_ref[...],
                               preferred_element_type=f32)

  return pl.pallas_call(
      body,
      grid=(_NEB,),
      in_specs=[_mat_spec(_BE, F), _mat_spec(_BE, F), _mat_spec(_BE, R),
                _full_spec((F, F)), _full_spec((F,)),
                _full_spec((F, F)), _full_spec((F,)),
                _full_spec((F, F)), _full_spec((F,)),
                _full_spec((F, F)), _full_spec((F,)),
                _full_spec((F, F)), _full_spec((F,)),
                _full_spec((R, F))],
      out_specs=[_mat_spec(_BE, F), _mat_spec(_BE, F)],
      out_shape=[jax.ShapeDtypeStruct((N_EDGES, F), f32),
                 jax.ShapeDtypeStruct((N_EDGES, F), f32)],
  )(agg, m, rbf, dw, db, *rp, rbf_wn)


_BA = 1000
_NAB = N_ATOMS // _BA


def _tc_atom_out(parts, outp):
  ws = []
  for pb in outp:
    ws += [pb['w1'], pb['b1'], pb['w2'], pb['b2']]

  def body(p0_ref, p1_ref, p2_ref,
           w10, b10, w20, b20, w11, b11, w21, b21, w12, b12, w22, b22,
           o_ref):
    @pl.when(pl.program_id(0) == 0)
    def _():
      o_ref[...] = jnp.zeros((1, 1), f32)

    tot = jnp.zeros((), f32)
    for (p_ref, w1, b1, w2, b2) in (
        (p0_ref, w10, b10, w20, b20),
        (p1_ref, w11, b11, w21, b21),
        (p2_ref, w12, b12, w22, b22)):
      af = p_ref[0] + p_ref[1]
      h = _swish(af)
      h2 = _swish(jnp.dot(h, w1[...], preferred_element_type=f32)
                  + b1[...][None, :])
      o = jnp.dot(h2, w2[...], preferred_element_type=f32) + b2[...][None, :]
      tot = tot + jnp.sum(o)
    o_ref[...] = o_ref[...] + tot.reshape(1, 1)

  pspec = pl.BlockSpec((NC, _BA, F), lambda k: (0, k, 0))
  return pl.pallas_call(
      body,
      grid=(_NAB,),
      in_specs=[pspec, pspec, pspec,
                _full_spec((F, F // 2)), _full_spec((F // 2,)),
                _full_spec((F // 2, 1)), _full_spec((1,)),
                _full_spec((F, F // 2)), _full_spec((F // 2,)),
                _full_spec((F // 2, 1)), _full_spec((1,)),
                _full_spec((F, F // 2)), _full_spec((F // 2,)),
                _full_spec((F // 2, 1)), _full_spec((1,))],
      out_specs=_full_spec((1, 1)),
      out_shape=jax.ShapeDtypeStruct((1, 1), f32),
  )(*parts, *ws)


def kernel(z, pos, i, j, idx_kj, idx_ji, params):
  del z
  pxa, pya, pza = pos[:, 0], pos[:, 1], pos[:, 2]

  d2 = _sc_edge_d2(pxa, pya, pza, i, j)
  dot, n1, n2, d2kj = _sc_triplet_geom(pxa, pya, pza, i, j, idx_kj, idx_ji, d2)

  emb = params['emb']
  cvec = (emb['atom'][0] * emb['atom'][0]).astype(f32)
  outp = params['out']
  rbf, m, t0 = _tc_edge(d2, cvec, emb['rbf_w'], emb['rbf_b'], outp[0]['rbf_w'])

  ib0, ib1 = params['inter'][0], params['inter'][1]
  sf0, sf1 = _tc_triplet(dot, n1, n2, d2kj,
                         ib0['sbf_w'], ib0['sbf_b'], ib1['sbf_w'], ib1['sbf_b'])

  parts = [_sc_scatter_atoms(t0, i)]
  sfs = (sf0, sf1)
  for blk, ib in enumerate((ib0, ib1)):
    mkj = _tc_mkj(m, ib['m_w'], ib['m_b'])
    x2 = _sc_row_gather(mkj, idx_kj)
    bmat = jnp.transpose(ib['bil_w'], (2, 1, 0)).reshape(F, 8 * F)
    bil = _tc_bil(x2, sfs[blk], bmat, ib['bil_b'])
    agg = _sc_scatter_edges(bil, idx_ji)
    m, tn = _tc_update(agg, m, rbf, ib['down_w'], ib['down_b'], ib['res'],
                       outp[blk + 1]['rbf_w'])
    parts.append(_sc_scatter_atoms(tn, i))

  total = _tc_atom_out(parts, outp)
  return total[0, 0]


# speedup vs baseline: 1.8422x; 1.6881x over previous
import functools

import jax
import jax.numpy as jnp
import numpy as np
from jax import lax
from jax.experimental import pallas as pl
from jax.experimental.pallas import tpu as pltpu
from jax.experimental.pallas import tpu_sc as plsc

N_ATOMS = 10000
N_EDGES = 160000
N_TRIPLETS = 160000
F = 128
R = 6
S = 7
CUTOFF = 5.0
P = 7

NC, NS, L = 2, 16, 16
NW = NC * NS
EPW = N_EDGES // NW
TPW = N_TRIPLETS // NW
TPS = N_TRIPLETS // NS

def _sc_mesh():
  return plsc.VectorSubcoreMesh(
      core_axis_name="c", subcore_axis_name="s", num_cores=NC, num_subcores=NS)

f32 = jnp.float32
i32 = jnp.int32


def _swish(x):
  return x * jax.nn.sigmoid(x)


def _envelope(x):
  p = P
  a = -(p + 1) * (p + 2) / 2.0
  b = p * (p + 2)
  c = -p * (p + 1) / 2.0
  xp0 = x ** (p - 1)
  xp1 = xp0 * x
  xp2 = xp1 * x
  return 1.0 / x + a * xp0 + b * xp1 + c * xp2


def _wid():
  return lax.axis_index("c") * NS + lax.axis_index("s")


def _sc_edge_d2(pxa, pya, pza, i_arr, j_arr):
  @functools.partial(
      pl.kernel,
      out_type=jax.ShapeDtypeStruct((N_EDGES,), f32),
      mesh=_sc_mesh(),
      compiler_params=pltpu.CompilerParams(needs_layout_passes=False),
      scratch_types=[
          pltpu.VMEM((N_ATOMS,), f32),
          pltpu.VMEM((N_ATOMS,), f32),
          pltpu.VMEM((N_ATOMS,), f32),
          pltpu.VMEM((EPW,), i32),
          pltpu.VMEM((EPW,), i32),
          pltpu.VMEM((EPW,), f32),
      ],
  )
  def k(px_hbm, py_hbm, pz_hbm, i_hbm, j_hbm, d2_hbm, px, py, pz, iv, jv, ov):
    w = _wid()
    base = w * EPW
    pltpu.sync_copy(px_hbm, px)
    pltpu.sync_copy(py_hbm, py)
    pltpu.sync_copy(pz_hbm, pz)
    pltpu.sync_copy(i_hbm.at[pl.ds(base, EPW)], iv)
    pltpu.sync_copy(j_hbm.at[pl.ds(base, EPW)], jv)

    def body(g, _):
      st = jnp.minimum(g * L, EPW - L)
      a16 = iv[pl.ds(st, L)]
      b16 = jv[pl.ds(st, L)]
      dx = plsc.load_gather(px, [a16]) - plsc.load_gather(px, [b16])
      dy = plsc.load_gather(py, [a16]) - plsc.load_gather(py, [b16])
      dz = plsc.load_gather(pz, [a16]) - plsc.load_gather(pz, [b16])
      ov[pl.ds(st, L)] = dx * dx + dy * dy + dz * dz
      return 0

    lax.fori_loop(0, (EPW + L - 1) // L, body, 0)
    pltpu.sync_copy(ov, d2_hbm.at[pl.ds(base, EPW)])

  return k(pxa, pya, pza, i_arr, j_arr)


def _sc_triplet_geom(pxa, pya, pza, i_arr, j_arr, idx_kj, idx_ji, d2):
  out_t = tuple(jax.ShapeDtypeStruct((N_TRIPLETS,), f32) for _ in range(4))

  @functools.partial(
      pl.kernel,
      out_type=out_t,
      mesh=_sc_mesh(),
      compiler_params=pltpu.CompilerParams(needs_layout_passes=False),
      scratch_types=[
          pltpu.VMEM((N_ATOMS,), f32),
          pltpu.VMEM((N_ATOMS,), f32),
          pltpu.VMEM((N_ATOMS,), f32),
          pltpu.VMEM((TPW,), i32),
          pltpu.VMEM((TPW,), i32),
          pltpu.VMEM((TPW,), i32),
          pltpu.VMEM((TPW,), i32),
          pltpu.VMEM((TPW,), i32),
          pltpu.VMEM((TPW,), f32),
          pltpu.VMEM((TPW,), f32),
          pltpu.VMEM((TPW,), f32),
          pltpu.VMEM((TPW,), f32),
          pltpu.SemaphoreType.DMA,
      ],
  )
  def k(px_hbm, py_hbm, pz_hbm, i_hbm, j_hbm, kj_hbm, ji_hbm, d2_hbm,
        dot_hbm, n1_hbm, n2_hbm, d2kj_hbm,
        px, py, pz, jiv, kjv, av, bv, kv, dkv, odot, on1, on2, sem):
    w = _wid()
    base = w * TPW
    pltpu.sync_copy(px_hbm, px)
    pltpu.sync_copy(py_hbm, py)
    pltpu.sync_copy(pz_hbm, pz)
    pltpu.sync_copy(ji_hbm.at[pl.ds(base, TPW)], jiv)
    pltpu.sync_copy(kj_hbm.at[pl.ds(base, TPW)], kjv)
    pltpu.async_copy(i_hbm.at[jiv], av, sem).wait()
    pltpu.async_copy(j_hbm.at[jiv], bv, sem).wait()
    pltpu.async_copy(j_hbm.at[kjv], kv, sem).wait()
    pltpu.async_copy(d2_hbm.at[kjv], dkv, sem).wait()

    def body(g, _):
      st = jnp.minimum(g * L, TPW - L)
      a16 = av[pl.ds(st, L)]
      b16 = bv[pl.ds(st, L)]
      k16 = kv[pl.ds(st, L)]
      bx = plsc.load_gather(px, [b16])
      by = plsc.load_gather(py, [b16])
      bz = plsc.load_gather(pz, [b16])
      v1x = plsc.load_gather(px, [a16]) - bx
      v1y = plsc.load_gather(py, [a16]) - by
      v1z = plsc.load_gather(pz, [a16]) - bz
      v2x = plsc.load_gather(px, [k16]) - bx
      v2y = plsc.load_gather(py, [k16]) - by
      v2z = plsc.load_gather(pz, [k16]) - bz
      odot[pl.ds(st, L)] = v1x * v2x + v1y * v2y + v1z * v2z
      on1[pl.ds(st, L)] = v1x * v1x + v1y * v1y + v1z * v1z
      on2[pl.ds(st, L)] = v2x * v2x + v2y * v2y + v2z * v2z
      return 0

    lax.fori_loop(0, (TPW + L - 1) // L, body, 0)
    pltpu.sync_copy(odot, dot_hbm.at[pl.ds(base, TPW)])
    pltpu.sync_copy(on1, n1_hbm.at[pl.ds(base, TPW)])
    pltpu.sync_copy(on2, n2_hbm.at[pl.ds(base, TPW)])
    pltpu.sync_copy(dkv, d2kj_hbm.at[pl.ds(base, TPW)])

  return k(pxa, pya, pza, i_arr, j_arr, idx_kj, idx_ji, d2)


_G_CHUNK = 200


def _sc_row_gather(tab, idx):
  n_chunks = TPW // _G_CHUNK

  @functools.partial(
      pl.kernel,
      out_type=jax.ShapeDtypeStruct((N_TRIPLETS, F), f32),
      mesh=_sc_mesh(),
      compiler_params=pltpu.CompilerParams(needs_layout_passes=False),
      scratch_types=[
          pltpu.VMEM((_G_CHUNK,), i32),
          pltpu.VMEM((_G_CHUNK, F), f32),
          pltpu.SemaphoreType.DMA,
      ],
  )
  def k(tab_hbm, idx_hbm, out_hbm, idxv, rows, sem):
    w = _wid()
    base = w * TPW

    def body(ci, _):
      off = pl.multiple_of(base + ci * _G_CHUNK, 8)
      pltpu.sync_copy(idx_hbm.at[pl.ds(off, _G_CHUNK)], idxv)
      pltpu.async_copy(tab_hbm.at[idxv], rows, sem).wait()
      pltpu.sync_copy(rows, out_hbm.at[pl.ds(off, _G_CHUNK)])
      return 0

    lax.fori_loop(0, n_chunks, body, 0)

  return k(tab, idx)


_CH = 8000
_CHP = 8192
_EPC = N_EDGES // NC
_NCHUNK = _EPC // _CH
_ZROWS = _CHP // NS
_WROWS = 1000


def _sc_scatter_edges(rows_hbm_arr, idx):
  @functools.partial(
      pl.kernel,
      out_type=jax.ShapeDtypeStruct((N_EDGES, F), f32),
      mesh=_sc_mesh(),
      compiler_params=pltpu.CompilerParams(needs_layout_passes=False),
      scratch_types=[
          pltpu.VMEM((TPS,), i32),
          pltpu.VMEM((TPS + 9 * L,), i32),
          pltpu.VMEM((TPS + 9 * L,), i32),
          pltpu.VMEM((4 * L, F), f32),
          pltpu.VMEM((L, F), f32),
          pltpu.VMEM_SHARED((_CHP, F), f32),
          pltpu.SemaphoreType.DMA,
          pltpu.SemaphoreType.DMA,
      ],
  )
  def k(rows_hbm, idx_hbm, agg_hbm, idxv, tlist, llist, rbuf, zrow, acc,
        sem, sem2):
    c = lax.axis_index("c")
    s = lax.axis_index("s")
    sbase = s * TPS
    pltpu.sync_copy(idx_hbm.at[pl.ds(sbase, TPS)], idxv)

    def zfill(r, _):
      for gg in range(F // L):
        zrow[r, pl.ds(gg * L, L)] = jnp.zeros((L,), f32)
      return 0

    lax.fori_loop(0, L, zfill, 0)

    def chunk_body(ci, _):
      cbase = c * _EPC + ci * _CH
      def zcopy(r, _):
        pltpu.sync_copy(
            zrow, acc.at[pl.ds(pl.multiple_of(s * _ZROWS + r * L, 8), L)])
        return 0

      lax.fori_loop(0, _ZROWS // L, zcopy, 0)
      plsc.subcore_barrier()

      def scan_body(g, ptr):
        v = idxv[pl.ds(g * L, L)]
        local = v - cbase
        m = (local >= 0) & (local < _CH)
        tglob = lax.iota(i32, L) + (sbase + g * L)
        plsc.store_compressed(tlist.at[pl.ds(ptr, L)], tglob, mask=m)
        plsc.store_compressed(llist.at[pl.ds(ptr, L)], local, mask=m)
        return ptr + jnp.sum(jnp.where(m, 1, 0).astype(i32))

      ptr = lax.fori_loop(0, TPS // L, scan_body, jnp.int32(0))
      for kk in range(8):
        tlist[pl.ds(ptr + kk * L, L)] = jnp.zeros((L,), i32)
        llist[pl.ds(ptr + kk * L, L)] = jnp.full((L,), _CH, i32)

      nb = (ptr + (L - 1)) // L

      def batch_body(b, _):
        tvec = tlist[pl.ds(b * L, L)]
        lvec = llist[pl.ds(b * L, L)]
        pltpu.async_copy(rows_hbm.at[tvec], rbuf.at[pl.ds(0, L)], sem).wait()
        pltpu.sync_copy(rbuf.at[pl.ds(0, L)], acc.at[lvec], add=True)
        return 0

      lax.fori_loop(0, nb, batch_body, 0)
      plsc.subcore_barrier()
      @pl.when(s < 8)
      def _():
        pltpu.sync_copy(
            acc.at[pl.ds(pl.multiple_of(s * _WROWS, 8), _WROWS)],
            agg_hbm.at[pl.ds(pl.multiple_of(cbase + s * _WROWS, 8), _WROWS)])
      plsc.subcore_barrier()
      return 0

    lax.fori_loop(0, _NCHUNK, chunk_body, 0)

  return k(rows_hbm_arr, idx)


_NP_PAD = 10240
_AZROWS = _NP_PAD // NS


def _sc_scatter_atoms(rows_hbm_arr, i_arr):
  n_full = EPW // L
  tail_valid = EPW - n_full * L

  @functools.partial(
      pl.kernel,
      out_type=jax.ShapeDtypeStruct((NC, _NP_PAD, F), f32),
      mesh=_sc_mesh(),
      compiler_params=pltpu.CompilerParams(needs_layout_passes=False),
      scratch_types=[
          pltpu.VMEM((EPW,), i32),
          pltpu.VMEM((4 * L, F), f32),
          pltpu.VMEM((4 * L, F), f32),
          pltpu.VMEM((L, F), f32),
          pltpu.VMEM_SHARED((_NP_PAD, F), f32),
          pltpu.SemaphoreType.DMA,
          pltpu.SemaphoreType.DMA,
          pltpu.SemaphoreType.DMA,
      ],
  )
  def k(rows_hbm, i_hbm, part_hbm, iv, rbufa, rbufb, zrow, acc,
        sema, semb, sems):
    c = lax.axis_index("c")
    s = lax.axis_index("s")
    w = c * NS + s
    base = w * EPW
    pltpu.sync_copy(i_hbm.at[pl.ds(base, EPW)], iv)

    def zfill(r, _):
      for gg in range(F // L):
        zrow[r, pl.ds(gg * L, L)] = jnp.zeros((L,), f32)
      return 0

    lax.fori_loop(0, L, zfill, 0)

    def zcopy(r, _):
      pltpu.sync_copy(
          zrow, acc.at[pl.ds(pl.multiple_of(s * _AZROWS + r * L, 8), L)])
      return 0

    lax.fori_loop(0, _AZROWS // L, zcopy, 0)
    plsc.subcore_barrier()

    lane = lax.iota(i32, L)
    n_pair = n_full // 8

    def body(m, _):
      st_a = m * (8 * L)
      st_b = st_a + 4 * L
      lda = pltpu.async_copy(
          rows_hbm.at[pl.ds(pl.multiple_of(base + st_a, 8), 4 * L)],
          rbufa, sema)
      ldb = pltpu.async_copy(
          rows_hbm.at[pl.ds(pl.multiple_of(base + st_b, 8), 4 * L)],
          rbufb, semb)
      sds = []
      lda.wait()
      for kk in range(4):
        ivec = iv[pl.ds(st_a + kk * L, L)]
        sds.append(pltpu.async_copy(
            rbufa.at[pl.ds(kk * L, L)], acc.at[ivec], sems, add=True))
      ldb.wait()
      for kk in range(4):
        ivec = iv[pl.ds(st_b + kk * L, L)]
        sds.append(pltpu.async_copy(
            rbufb.at[pl.ds(kk * L, L)], acc.at[ivec], sems, add=True))
      for d in sds:
        d.wait()
      return 0

    lax.fori_loop(0, n_pair, body, 0)
    for g in range(n_pair * 8, n_full):
      st = g * L
      pltpu.sync_copy(
          rows_hbm.at[pl.ds(pl.multiple_of(base + st, 8), L)],
          rbufa.at[pl.ds(0, L)])
      ivec = iv[pl.ds(st, L)]
      pltpu.sync_copy(rbufa.at[pl.ds(0, L)], acc.at[ivec], add=True)
    if tail_valid:
      st = EPW - L
      pltpu.sync_copy(rows_hbm.at[pl.ds(pl.multiple_of(base + st, 8), L)],
                      rbufa.at[pl.ds(0, L)])
      ivec = iv[pl.ds(st, L)]
      ivec = jnp.where(lane < (L - tail_valid), jnp.full((L,), N_ATOMS, i32),
                       ivec)
      pltpu.sync_copy(rbufa.at[pl.ds(0, L)], acc.at[ivec], add=True)

    plsc.subcore_barrier()
    pltpu.sync_copy(
        acc.at[pl.ds(pl.multiple_of(s * _AZROWS, 8), _AZROWS)],
        part_hbm.at[c, pl.ds(pl.multiple_of(s * _AZROWS, 8), _AZROWS)])

  return k(rows_hbm_arr, i_arr)


_BE = 256
_NEB = N_EDGES // _BE
_BE2 = 2000
_NEB2 = N_EDGES // _BE2

def _freq_row():
  return (lax.broadcasted_iota(jnp.int32, (1, R), 1).astype(f32) + 1.0) * np.pi


def _sbf_consts():
  q = lax.broadcasted_iota(jnp.int32, (1, S * R), 1)
  r = (q % R).astype(f32)
  s = (q // R).astype(f32)
  bwf = (r + 1.0) * np.pi + s * (S - 1) * np.pi
  return bwf, s


def _vec_spec(b):
  return pl.BlockSpec((b,), lambda k: (k,))


def _mat_spec(b, d):
  return pl.BlockSpec((b, d), lambda k: (k, 0))


def _full_spec(shape):
  nd = len(shape)
  return pl.BlockSpec(shape, lambda k: (0,) * nd)


def _tc_edge(d2, cvec, emb_w, emb_b, rbf_w0):
  def body(d2_ref, c_ref, ew_ref, eb_ref, rw0_ref, rbf_ref, m0_ref, t0_ref):
    d2t = d2_ref[...]
    dist = jnp.sqrt(d2t)
    ds_ = dist / CUTOFF
    env = _envelope(ds_)
    rbf = env[:, None] * jnp.sin(_freq_row() * ds_[:, None]) / dist[:, None]
    rbf_ref[...] = rbf
    m0 = c_ref[...][None, :] * _swish(
        jnp.dot(rbf, ew_ref[...], preferred_element_type=f32)
        + eb_ref[...][None, :])
    m0_ref[...] = m0
    t0_ref[...] = m0 * jnp.dot(rbf, rw0_ref[...], preferred_element_type=f32)

  return pl.pallas_call(
      body,
      grid=(_NEB,),
      in_specs=[_vec_spec(_BE), _full_spec((F,)), _full_spec((R, F)),
                _full_spec((F,)), _full_spec((R, F))],
      out_specs=[_mat_spec(_BE, R), _mat_spec(_BE, F), _mat_spec(_BE, F)],
      out_shape=[jax.ShapeDtypeStruct((N_EDGES, R), f32),
                 jax.ShapeDtypeStruct((N_EDGES, F), f32),
                 jax.ShapeDtypeStruct((N_EDGES, F), f32)],
  )(d2, cvec, emb_w, emb_b, rbf_w0)


def _tc_triplet(dot, n1, n2, d2kj, sw0, sb0, sw1, sb1):
  def body(dot_ref, n1_ref, n2_ref, dk_ref, w0_ref, b0_ref, w1_ref, b1_ref,
           sf0_ref, sf1_ref):
    dott = dot_ref[...]
    nrm = jnp.sqrt(n1_ref[...] + 1e-12) * jnp.sqrt(n2_ref[...] + 1e-12)
    ca = jnp.clip(dott / (nrm + 1e-7), -1.0 + 1e-6, 1.0 - 1e-6)
    dk = jnp.sqrt(dk_ref[...]) / CUTOFF
    env = _envelope(dk)
    bwf, shwf = _sbf_consts()
    cprev = jnp.ones_like(ca)
    ccur = ca
    sph = (shwf == 0.0).astype(f32) * cprev[:, None]
    sph = sph + (shwf == 1.0).astype(f32) * ccur[:, None]
    for s_ in range(2, S):
      cnext = 2.0 * ca * ccur - cprev
      cprev, ccur = ccur, cnext
      sph = sph + (shwf == float(s_)).astype(f32) * ccur[:, None]
    sbf = env[:, None] * jnp.sin(bwf * dk[:, None]) * sph
    sf0_ref[...] = _swish(
        jnp.dot(sbf, w0_ref[...], preferred_element_type=f32)
        + b0_ref[...][None, :])
    sf1_ref[...] = _swish(
        jnp.dot(sbf, w1_ref[...], preferred_element_type=f32)
        + b1_ref[...][None, :])

  return pl.pallas_call(
      body,
      grid=(_NEB,),
      in_specs=[_vec_spec(_BE)] * 4
      + [_full_spec((S * R, 8)), _full_spec((8,)),
         _full_spec((S * R, 8)), _full_spec((8,))],
      out_specs=[_mat_spec(_BE, 8), _mat_spec(_BE, 8)],
      out_shape=[jax.ShapeDtypeStruct((N_TRIPLETS, 8), f32),
                 jax.ShapeDtypeStruct((N_TRIPLETS, 8), f32)],
  )(dot, n1, n2, d2kj, sw0, sb0, sw1, sb1)


def _tc_mkj(m, mw, mb):
  def body(m_ref, w_ref, b_ref, o_ref):
    o_ref[...] = _swish(
        jnp.dot(m_ref[...], w_ref[...], preferred_element_type=f32)
        + b_ref[...][None, :])

  return pl.pallas_call(
      body,
      grid=(_NEB2,),
      in_specs=[_mat_spec(_BE2, F), _full_spec((F, F)), _full_spec((F,))],
      out_specs=_mat_spec(_BE2, F),
      out_shape=jax.ShapeDtypeStruct((N_EDGES, F), f32),
  )(m, mw, mb)


def _tc_bil(x2, sf, bmat, bil_b):
  def body(x_ref, sf_ref, bm_ref, bb_ref, o_ref):
    y = jnp.dot(x_ref[...], bm_ref[...], preferred_element_type=f32)
    sft = sf_ref[...]
    acc = bb_ref[...][None, :]
    for ii in range(8):
      acc = acc + sft[:, ii][:, None] * y[:, ii * F:(ii + 1) * F]
    o_ref[...] = acc

  return pl.pallas_call(
      body,
      grid=(_NEB2,),
      in_specs=[_mat_spec(_BE2, F), _mat_spec(_BE2, 8),
                _full_spec((F, 8 * F)), _full_spec((F,))],
      out_specs=_mat_spec(_BE2, F),
      out_shape=jax.ShapeDtypeStruct((N_TRIPLETS, F), f32),
  )(x2, sf, bmat, bil_b)


def _tc_update(agg, m, rbf, dw, db, res, rbf_wn):
  rp = [res[0]['w1'], res[0]['b1'], res[0]['w2'], res[0]['b2'],
        res[1]['w1'], res[1]['b1'], res[1]['w2'], res[1]['b2']]

  def body(agg_ref, m_ref, rbf_ref, dw_ref, db_ref,
           r0w1, r0b1, r0w2, r0b2, r1w1, r1b1, r1w2, r1b2, rwn_ref,
           mn_ref, tn_ref):
    mm = m_ref[...] + _swish(
        jnp.dot(agg_ref[...], dw_ref[...], preferred_element_type=f32)
        + db_ref[...][None, :])
    for (w1, b1, w2, b2) in ((r0w1, r0b1, r0w2, r0b2),
                             (r1w1, r1b1, r1w2, r1b2)):
      h = _swish(jnp.dot(mm, w1[...], preferred_element_type=f32)
                 + b1[...][None, :])
      mm = mm + jnp.dot(h, w2[...], preferred_element_type=f32) + b2[...][None, :]
    mn_ref[...] = mm
    tn_ref[...] = mm * jnp.dot(rbf_ref[...], rwn_ref[...],
                               preferred_element_type=f32)

  return pl.pallas_call(
      body,
      grid=(_NEB2,),
      in_specs=[_mat_spec(_BE2, F), _mat_spec(_BE2, F), _mat_spec(_BE2, R),
                _full_spec((F, F)), _full_spec((F,)),
                _full_spec((F, F)), _full_spec((F,)),
                _full_spec((F, F)), _full_spec((F,)),
                _full_spec((F, F)), _full_spec((F,)),
                _full_spec((F, F)), _full_spec((F,)),
                _full_spec((R, F))],
      out_specs=[_mat_spec(_BE2, F), _mat_spec(_BE2, F)],
      out_shape=[jax.ShapeDtypeStruct((N_EDGES, F), f32),
                 jax.ShapeDtypeStruct((N_EDGES, F), f32)],
  )(agg, m, rbf, dw, db, *rp, rbf_wn)


_BA = 1000
_NAB = N_ATOMS // _BA


def _tc_atom_out(parts, outp):
  ws = []
  for pb in outp:
    ws += [pb['w1'], pb['b1'], pb['w2'], pb['b2']]

  def body(p0_ref, p1_ref, p2_ref,
           w10, b10, w20, b20, w11, b11, w21, b21, w12, b12, w22, b22,
           o_ref):
    @pl.when(pl.program_id(0) == 0)
    def _():
      o_ref[...] = jnp.zeros((1, 1), f32)

    tot = jnp.zeros((), f32)
    for (p_ref, w1, b1, w2, b2) in (
        (p0_ref, w10, b10, w20, b20),
        (p1_ref, w11, b11, w21, b21),
        (p2_ref, w12, b12, w22, b22)):
      af = p_ref[0] + p_ref[1]
      h = _swish(af)
      h2 = _swish(jnp.dot(h, w1[...], preferred_element_type=f32)
                  + b1[...][None, :])
      o = jnp.dot(h2, w2[...], preferred_element_type=f32) + b2[...][None, :]
      tot = tot + jnp.sum(o)
    o_ref[...] = o_ref[...] + tot.reshape(1, 1)

  pspec = pl.BlockSpec((NC, _BA, F), lambda k: (0, k, 0))
  return pl.pallas_call(
      body,
      grid=(_NAB,),
      in_specs=[pspec, pspec, pspec,
                _full_spec((F, F // 2)), _full_spec((F // 2,)),
                _full_spec((F // 2, 1)), _full_spec((1,)),
                _full_spec((F, F // 2)), _full_spec((F // 2,)),
                _full_spec((F // 2, 1)), _full_spec((1,)),
                _full_spec((F, F // 2)), _full_spec((F // 2,)),
                _full_spec((F // 2, 1)), _full_spec((1,))],
      out_specs=_full_spec((1, 1)),
      out_shape=jax.ShapeDtypeStruct((1, 1), f32),
  )(*parts, *ws)


def kernel(z, pos, i, j, idx_kj, idx_ji, params):
  del z
  pxa, pya, pza = pos[:, 0], pos[:, 1], pos[:, 2]

  d2 = _sc_edge_d2(pxa, pya, pza, i, j)
  dot, n1, n2, d2kj = _sc_triplet_geom(pxa, pya, pza, i, j, idx_kj, idx_ji, d2)

  emb = params['emb']
  cvec = (emb['atom'][0] * emb['atom'][0]).astype(f32)
  outp = params['out']
  rbf, m, t0 = _tc_edge(d2, cvec, emb['rbf_w'], emb['rbf_b'], outp[0]['rbf_w'])

  ib0, ib1 = params['inter'][0], params['inter'][1]
  sf0, sf1 = _tc_triplet(dot, n1, n2, d2kj,
                         ib0['sbf_w'], ib0['sbf_b'], ib1['sbf_w'], ib1['sbf_b'])

  parts = [_sc_scatter_atoms(t0, i)]
  sfs = (sf0, sf1)
  for blk, ib in enumerate((ib0, ib1)):
    mkj = _tc_mkj(m, ib['m_w'], ib['m_b'])
    x2 = _sc_row_gather(mkj, idx_kj)
    bmat = jnp.transpose(ib['bil_w'], (2, 1, 0)).reshape(F, 8 * F)
    bil = _tc_bil(x2, sfs[blk], bmat, ib['bil_b'])
    agg = _sc_scatter_edges(bil, idx_ji)
    m, tn = _tc_update(agg, m, rbf, ib['down_w'], ib['down_b'], ib['res'],
                       outp[blk + 1]['rbf_w'])
    parts.append(_sc_scatter_atoms(tn, i))

  total = _tc_atom_out(parts, outp)
  return total[0, 0]


# speedup vs baseline: 1.8466x; 1.0024x over previous
import functools

import jax
import jax.numpy as jnp
import numpy as np
from jax import lax
from jax.experimental import pallas as pl
from jax.experimental.pallas import tpu as pltpu
from jax.experimental.pallas import tpu_sc as plsc

N_ATOMS = 10000
N_EDGES = 160000
N_TRIPLETS = 160000
F = 128
R = 6
S = 7
CUTOFF = 5.0
P = 7

NC, NS, L = 2, 16, 16
NW = NC * NS
EPW = N_EDGES // NW
TPW = N_TRIPLETS // NW
TPS = N_TRIPLETS // NS

def _sc_mesh():
  return plsc.VectorSubcoreMesh(
      core_axis_name="c", subcore_axis_name="s", num_cores=NC, num_subcores=NS)

f32 = jnp.float32
i32 = jnp.int32


def _swish(x):
  return x * jax.nn.sigmoid(x)


def _envelope(x):
  p = P
  a = -(p + 1) * (p + 2) / 2.0
  b = p * (p + 2)
  c = -p * (p + 1) / 2.0
  xp0 = x ** (p - 1)
  xp1 = xp0 * x
  xp2 = xp1 * x
  return 1.0 / x + a * xp0 + b * xp1 + c * xp2


def _wid():
  return lax.axis_index("c") * NS + lax.axis_index("s")


def _sc_edge_d2(pxa, pya, pza, i_arr, j_arr):
  @functools.partial(
      pl.kernel,
      out_type=jax.ShapeDtypeStruct((N_EDGES,), f32),
      mesh=_sc_mesh(),
      compiler_params=pltpu.CompilerParams(needs_layout_passes=False),
      scratch_types=[
          pltpu.VMEM((N_ATOMS,), f32),
          pltpu.VMEM((N_ATOMS,), f32),
          pltpu.VMEM((N_ATOMS,), f32),
          pltpu.VMEM((EPW,), i32),
          pltpu.VMEM((EPW,), i32),
          pltpu.VMEM((EPW,), f32),
      ],
  )
  def k(px_hbm, py_hbm, pz_hbm, i_hbm, j_hbm, d2_hbm, px, py, pz, iv, jv, ov):
    w = _wid()
    base = w * EPW
    pltpu.sync_copy(px_hbm, px)
    pltpu.sync_copy(py_hbm, py)
    pltpu.sync_copy(pz_hbm, pz)
    pltpu.sync_copy(i_hbm.at[pl.ds(base, EPW)], iv)
    pltpu.sync_copy(j_hbm.at[pl.ds(base, EPW)], jv)

    def body(g, _):
      st = jnp.minimum(g * L, EPW - L)
      a16 = iv[pl.ds(st, L)]
      b16 = jv[pl.ds(st, L)]
      dx = plsc.load_gather(px, [a16]) - plsc.load_gather(px, [b16])
      dy = plsc.load_gather(py, [a16]) - plsc.load_gather(py, [b16])
      dz = plsc.load_gather(pz, [a16]) - plsc.load_gather(pz, [b16])
      ov[pl.ds(st, L)] = dx * dx + dy * dy + dz * dz
      return 0

    lax.fori_loop(0, (EPW + L - 1) // L, body, 0)
    pltpu.sync_copy(ov, d2_hbm.at[pl.ds(base, EPW)])

  return k(pxa, pya, pza, i_arr, j_arr)


def _sc_triplet_geom(pxa, pya, pza, i_arr, j_arr, idx_kj, idx_ji, d2):
  out_t = tuple(jax.ShapeDtypeStruct((N_TRIPLETS,), f32) for _ in range(4))

  @functools.partial(
      pl.kernel,
      out_type=out_t,
      mesh=_sc_mesh(),
      compiler_params=pltpu.CompilerParams(needs_layout_passes=False),
      scratch_types=[
          pltpu.VMEM((N_ATOMS,), f32),
          pltpu.VMEM((N_ATOMS,), f32),
          pltpu.VMEM((N_ATOMS,), f32),
          pltpu.VMEM((TPW,), i32),
          pltpu.VMEM((TPW,), i32),
          pltpu.VMEM((TPW,), i32),
          pltpu.VMEM((TPW,), i32),
          pltpu.VMEM((TPW,), i32),
          pltpu.VMEM((TPW,), f32),
          pltpu.VMEM((TPW,), f32),
          pltpu.VMEM((TPW,), f32),
          pltpu.VMEM((TPW,), f32),
          pltpu.SemaphoreType.DMA,
      ],
  )
  def k(px_hbm, py_hbm, pz_hbm, i_hbm, j_hbm, kj_hbm, ji_hbm, d2_hbm,
        dot_hbm, n1_hbm, n2_hbm, d2kj_hbm,
        px, py, pz, jiv, kjv, av, bv, kv, dkv, odot, on1, on2, sem):
    w = _wid()
    base = w * TPW
    pltpu.sync_copy(px_hbm, px)
    pltpu.sync_copy(py_hbm, py)
    pltpu.sync_copy(pz_hbm, pz)
    pltpu.sync_copy(ji_hbm.at[pl.ds(base, TPW)], jiv)
    pltpu.sync_copy(kj_hbm.at[pl.ds(base, TPW)], kjv)
    pltpu.async_copy(i_hbm.at[jiv], av, sem).wait()
    pltpu.async_copy(j_hbm.at[jiv], bv, sem).wait()
    pltpu.async_copy(j_hbm.at[kjv], kv, sem).wait()
    pltpu.async_copy(d2_hbm.at[kjv], dkv, sem).wait()

    def body(g, _):
      st = jnp.minimum(g * L, TPW - L)
      a16 = av[pl.ds(st, L)]
      b16 = bv[pl.ds(st, L)]
      k16 = kv[pl.ds(st, L)]
      bx = plsc.load_gather(px, [b16])
      by = plsc.load_gather(py, [b16])
      bz = plsc.load_gather(pz, [b16])
      v1x = plsc.load_gather(px, [a16]) - bx
      v1y = plsc.load_gather(py, [a16]) - by
      v1z = plsc.load_gather(pz, [a16]) - bz
      v2x = plsc.load_gather(px, [k16]) - bx
      v2y = plsc.load_gather(py, [k16]) - by
      v2z = plsc.load_gather(pz, [k16]) - bz
      odot[pl.ds(st, L)] = v1x * v2x + v1y * v2y + v1z * v2z
      on1[pl.ds(st, L)] = v1x * v1x + v1y * v1y + v1z * v1z
      on2[pl.ds(st, L)] = v2x * v2x + v2y * v2y + v2z * v2z
      return 0

    lax.fori_loop(0, (TPW + L - 1) // L, body, 0)
    pltpu.sync_copy(odot, dot_hbm.at[pl.ds(base, TPW)])
    pltpu.sync_copy(on1, n1_hbm.at[pl.ds(base, TPW)])
    pltpu.sync_copy(on2, n2_hbm.at[pl.ds(base, TPW)])
    pltpu.sync_copy(dkv, d2kj_hbm.at[pl.ds(base, TPW)])

  return k(pxa, pya, pza, i_arr, j_arr, idx_kj, idx_ji, d2)


_G_CHUNK = 200


def _sc_row_gather(tab, idx):
  n_chunks = TPW // _G_CHUNK
  dt = tab.dtype
  dcols = tab.shape[1]

  @functools.partial(
      pl.kernel,
      out_type=jax.ShapeDtypeStruct((N_TRIPLETS, dcols), dt),
      mesh=_sc_mesh(),
      compiler_params=pltpu.CompilerParams(needs_layout_passes=False),
      scratch_types=[
          pltpu.VMEM((_G_CHUNK,), i32),
          pltpu.VMEM((_G_CHUNK, dcols), dt),
          pltpu.SemaphoreType.DMA,
      ],
  )
  def k(tab_hbm, idx_hbm, out_hbm, idxv, rows, sem):
    w = _wid()
    base = w * TPW

    def body(ci, _):
      off = pl.multiple_of(base + ci * _G_CHUNK, 8)
      pltpu.sync_copy(idx_hbm.at[pl.ds(off, _G_CHUNK)], idxv)
      pltpu.async_copy(tab_hbm.at[idxv], rows, sem).wait()
      pltpu.sync_copy(rows, out_hbm.at[pl.ds(off, _G_CHUNK)])
      return 0

    lax.fori_loop(0, n_chunks, body, 0)

  return k(tab, idx)


_CH = 8000
_CHP = 8192
_EPC = N_EDGES // NC
_NCHUNK = _EPC // _CH
_ZROWS = _CHP // NS
_WROWS = 1000


def _sc_scatter_edges(rows_hbm_arr, idx):
  @functools.partial(
      pl.kernel,
      out_type=jax.ShapeDtypeStruct((N_EDGES, F), f32),
      mesh=_sc_mesh(),
      compiler_params=pltpu.CompilerParams(needs_layout_passes=False),
      scratch_types=[
          pltpu.VMEM((TPS,), i32),
          pltpu.VMEM((TPS + 9 * L,), i32),
          pltpu.VMEM((TPS + 9 * L,), i32),
          pltpu.VMEM((4 * L, F), f32),
          pltpu.VMEM((L, F), f32),
          pltpu.VMEM_SHARED((_CHP, F), f32),
          pltpu.SemaphoreType.DMA,
          pltpu.SemaphoreType.DMA,
      ],
  )
  def k(rows_hbm, idx_hbm, agg_hbm, idxv, tlist, llist, rbuf, zrow, acc,
        sem, sem2):
    c = lax.axis_index("c")
    s = lax.axis_index("s")
    sbase = s * TPS
    pltpu.sync_copy(idx_hbm.at[pl.ds(sbase, TPS)], idxv)

    def zfill(r, _):
      for gg in range(F // L):
        zrow[r, pl.ds(gg * L, L)] = jnp.zeros((L,), f32)
      return 0

    lax.fori_loop(0, L, zfill, 0)

    def chunk_body(ci, _):
      cbase = c * _EPC + ci * _CH
      def zcopy(r, _):
        pltpu.sync_copy(
            zrow, acc.at[pl.ds(pl.multiple_of(s * _ZROWS + r * L, 8), L)])
        return 0

      lax.fori_loop(0, _ZROWS // L, zcopy, 0)
      plsc.subcore_barrier()

      def scan_body(g, ptr):
        v = idxv[pl.ds(g * L, L)]
        local = v - cbase
        m = (local >= 0) & (local < _CH)
        tglob = lax.iota(i32, L) + (sbase + g * L)
        plsc.store_compressed(tlist.at[pl.ds(ptr, L)], tglob, mask=m)
        plsc.store_compressed(llist.at[pl.ds(ptr, L)], local, mask=m)
        return ptr + jnp.sum(jnp.where(m, 1, 0).astype(i32))

      ptr = lax.fori_loop(0, TPS // L, scan_body, jnp.int32(0))
      for kk in range(8):
        tlist[pl.ds(ptr + kk * L, L)] = jnp.zeros((L,), i32)
        llist[pl.ds(ptr + kk * L, L)] = jnp.full((L,), _CH, i32)

      nb = (ptr + (L - 1)) // L

      def batch_body(b, _):
        tvec = tlist[pl.ds(b * L, L)]
        lvec = llist[pl.ds(b * L, L)]
        pltpu.async_copy(rows_hbm.at[tvec], rbuf.at[pl.ds(0, L)], sem).wait()
        pltpu.sync_copy(rbuf.at[pl.ds(0, L)], acc.at[lvec], add=True)
        return 0

      lax.fori_loop(0, nb, batch_body, 0)
      plsc.subcore_barrier()
      @pl.when(s < 8)
      def _():
        pltpu.sync_copy(
            acc.at[pl.ds(pl.multiple_of(s * _WROWS, 8), _WROWS)],
            agg_hbm.at[pl.ds(pl.multiple_of(cbase + s * _WROWS, 8), _WROWS)])
      plsc.subcore_barrier()
      return 0

    lax.fori_loop(0, _NCHUNK, chunk_body, 0)

  return k(rows_hbm_arr, idx)


_NP_PAD = 10240
_AZROWS = _NP_PAD // NS


def _sc_scatter_atoms(rows_hbm_arr, i_arr):
  n_full = EPW // L
  tail_valid = EPW - n_full * L

  @functools.partial(
      pl.kernel,
      out_type=jax.ShapeDtypeStruct((NC, _NP_PAD, F), f32),
      mesh=_sc_mesh(),
      compiler_params=pltpu.CompilerParams(needs_layout_passes=False),
      scratch_types=[
          pltpu.VMEM((EPW,), i32),
          pltpu.VMEM((4 * L, F), f32),
          pltpu.VMEM((4 * L, F), f32),
          pltpu.VMEM((L, F), f32),
          pltpu.VMEM_SHARED((_NP_PAD, F), f32),
          pltpu.SemaphoreType.DMA,
          pltpu.SemaphoreType.DMA,
          pltpu.SemaphoreType.DMA,
      ],
  )
  def k(rows_hbm, i_hbm, part_hbm, iv, rbufa, rbufb, zrow, acc,
        sema, semb, sems):
    c = lax.axis_index("c")
    s = lax.axis_index("s")
    w = c * NS + s
    base = w * EPW
    pltpu.sync_copy(i_hbm.at[pl.ds(base, EPW)], iv)

    def zfill(r, _):
      for gg in range(F // L):
        zrow[r, pl.ds(gg * L, L)] = jnp.zeros((L,), f32)
      return 0

    lax.fori_loop(0, L, zfill, 0)

    def zcopy(r, _):
      pltpu.sync_copy(
          zrow, acc.at[pl.ds(pl.multiple_of(s * _AZROWS + r * L, 8), L)])
      return 0

    lax.fori_loop(0, _AZROWS // L, zcopy, 0)
    plsc.subcore_barrier()

    lane = lax.iota(i32, L)
    n_pair = n_full // 8

    def body(m, _):
      st_a = m * (8 * L)
      st_b = st_a + 4 * L
      lda = pltpu.async_copy(
          rows_hbm.at[pl.ds(pl.multiple_of(base + st_a, 8), 4 * L)],
          rbufa, sema)
      ldb = pltpu.async_copy(
          rows_hbm.at[pl.ds(pl.multiple_of(base + st_b, 8), 4 * L)],
          rbufb, semb)
      sds = []
      lda.wait()
      for kk in range(4):
        ivec = iv[pl.ds(st_a + kk * L, L)]
        sds.append(pltpu.async_copy(
            rbufa.at[pl.ds(kk * L, L)], acc.at[ivec], sems, add=True))
      ldb.wait()
      for kk in range(4):
        ivec = iv[pl.ds(st_b + kk * L, L)]
        sds.append(pltpu.async_copy(
            rbufb.at[pl.ds(kk * L, L)], acc.at[ivec], sems, add=True))
      for d in sds:
        d.wait()
      return 0

    lax.fori_loop(0, n_pair, body, 0)
    for g in range(n_pair * 8, n_full):
      st = g * L
      pltpu.sync_copy(
          rows_hbm.at[pl.ds(pl.multiple_of(base + st, 8), L)],
          rbufa.at[pl.ds(0, L)])
      ivec = iv[pl.ds(st, L)]
      pltpu.sync_copy(rbufa.at[pl.ds(0, L)], acc.at[ivec], add=True)
    if tail_valid:
      st = EPW - L
      pltpu.sync_copy(rows_hbm.at[pl.ds(pl.multiple_of(base + st, 8), L)],
                      rbufa.at[pl.ds(0, L)])
      ivec = iv[pl.ds(st, L)]
      ivec = jnp.where(lane < (L - tail_valid), jnp.full((L,), N_ATOMS, i32),
                       ivec)
      pltpu.sync_copy(rbufa.at[pl.ds(0, L)], acc.at[ivec], add=True)

    plsc.subcore_barrier()
    pltpu.sync_copy(
        acc.at[pl.ds(pl.multiple_of(s * _AZROWS, 8), _AZROWS)],
        part_hbm.at[c, pl.ds(pl.multiple_of(s * _AZROWS, 8), _AZROWS)])

  return k(rows_hbm_arr, i_arr)


_BE = 256
_NEB = N_EDGES // _BE
_BE2 = 2000
_NEB2 = N_EDGES // _BE2

def _freq_row():
  return (lax.broadcasted_iota(jnp.int32, (1, R), 1).astype(f32) + 1.0) * np.pi


def _sbf_consts():
  q = lax.broadcasted_iota(jnp.int32, (1, S * R), 1)
  r = (q % R).astype(f32)
  s = (q // R).astype(f32)
  bwf = (r + 1.0) * np.pi + s * (S - 1) * np.pi
  return bwf, s


def _vec_spec(b):
  return pl.BlockSpec((b,), lambda k: (k,))


def _mat_spec(b, d):
  return pl.BlockSpec((b, d), lambda k: (k, 0))


def _full_spec(shape):
  nd = len(shape)
  return pl.BlockSpec(shape, lambda k: (0,) * nd)


def _tc_edge(d2, cvec, emb_w, emb_b, rbf_w0):
  def body(d2_ref, c_ref, ew_ref, eb_ref, rw0_ref, rbf_ref, m0_ref, t0_ref):
    d2t = d2_ref[...]
    dist = jnp.sqrt(d2t)
    ds_ = dist / CUTOFF
    env = _envelope(ds_)
    rbf = env[:, None] * jnp.sin(_freq_row() * ds_[:, None]) / dist[:, None]
    rbf_ref[...] = rbf
    m0 = c_ref[...][None, :] * _swish(
        jnp.dot(rbf, ew_ref[...], preferred_element_type=f32)
        + eb_ref[...][None, :])
    m0_ref[...] = m0
    t0_ref[...] = m0 * jnp.dot(rbf, rw0_ref[...], preferred_element_type=f32)

  return pl.pallas_call(
      body,
      grid=(_NEB,),
      in_specs=[_vec_spec(_BE), _full_spec((F,)), _full_spec((R, F)),
                _full_spec((F,)), _full_spec((R, F))],
      out_specs=[_mat_spec(_BE, R), _mat_spec(_BE, F), _mat_spec(_BE, F)],
      out_shape=[jax.ShapeDtypeStruct((N_EDGES, R), f32),
                 jax.ShapeDtypeStruct((N_EDGES, F), f32),
                 jax.ShapeDtypeStruct((N_EDGES, F), f32)],
  )(d2, cvec, emb_w, emb_b, rbf_w0)


def _tc_triplet(dot, n1, n2, d2kj, sw0, sb0, sw1, sb1):
  def body(dot_ref, n1_ref, n2_ref, dk_ref, w0_ref, b0_ref, w1_ref, b1_ref,
           sf0_ref, sf1_ref):
    dott = dot_ref[...]
    nrm = jnp.sqrt(n1_ref[...] + 1e-12) * jnp.sqrt(n2_ref[...] + 1e-12)
    ca = jnp.clip(dott / (nrm + 1e-7), -1.0 + 1e-6, 1.0 - 1e-6)
    dk = jnp.sqrt(dk_ref[...]) / CUTOFF
    env = _envelope(dk)
    bwf, shwf = _sbf_consts()
    cprev = jnp.ones_like(ca)
    ccur = ca
    sph = (shwf == 0.0).astype(f32) * cprev[:, None]
    sph = sph + (shwf == 1.0).astype(f32) * ccur[:, None]
    for s_ in range(2, S):
      cnext = 2.0 * ca * ccur - cprev
      cprev, ccur = ccur, cnext
      sph = sph + (shwf == float(s_)).astype(f32) * ccur[:, None]
    sbf = env[:, None] * jnp.sin(bwf * dk[:, None]) * sph
    sf0_ref[...] = _swish(
        jnp.dot(sbf, w0_ref[...], preferred_element_type=f32)
        + b0_ref[...][None, :])
    sf1_ref[...] = _swish(
        jnp.dot(sbf, w1_ref[...], preferred_element_type=f32)
        + b1_ref[...][None, :])

  return pl.pallas_call(
      body,
      grid=(_NEB,),
      in_specs=[_vec_spec(_BE)] * 4
      + [_full_spec((S * R, 8)), _full_spec((8,)),
         _full_spec((S * R, 8)), _full_spec((8,))],
      out_specs=[_mat_spec(_BE, 8), _mat_spec(_BE, 8)],
      out_shape=[jax.ShapeDtypeStruct((N_TRIPLETS, 8), f32),
                 jax.ShapeDtypeStruct((N_TRIPLETS, 8), f32)],
  )(dot, n1, n2, d2kj, sw0, sb0, sw1, sb1)


def _tc_mkj(m, mw, mb):
  def body(m_ref, w_ref, b_ref, o_ref):
    o_ref[...] = _swish(
        jnp.dot(m_ref[...], w_ref[...], preferred_element_type=f32)
        + b_ref[...][None, :])

  return pl.pallas_call(
      body,
      grid=(_NEB2,),
      in_specs=[_mat_spec(_BE2, F), _full_spec((F, F)), _full_spec((F,))],
      out_specs=_mat_spec(_BE2, F),
      out_shape=jax.ShapeDtypeStruct((N_EDGES, F), f32),
  )(m, mw, mb)


def _tc_bil(x2, sf, bmat, bil_b):
  def body(x_ref, sf_ref, bm_ref, bb_ref, o_ref):
    y = jnp.dot(x_ref[...].astype(jnp.bfloat16), bm_ref[...],
                preferred_element_type=f32)
    sft = sf_ref[...]
    acc = bb_ref[...][None, :]
    for ii in range(8):
      acc = acc + sft[:, ii][:, None] * y[:, ii * F:(ii + 1) * F]
    o_ref[...] = acc

  return pl.pallas_call(
      body,
      grid=(_NEB2,),
      in_specs=[_mat_spec(_BE2, F), _mat_spec(_BE2, 8),
                _full_spec((F, 8 * F)), _full_spec((F,))],
      out_specs=_mat_spec(_BE2, F),
      out_shape=jax.ShapeDtypeStruct((N_TRIPLETS, F), f32),
  )(x2, sf, bmat.astype(jnp.bfloat16), bil_b)


def _tc_update(agg, m, rbf, dw, db, res, rbf_wn):
  rp = [res[0]['w1'], res[0]['b1'], res[0]['w2'], res[0]['b2'],
        res[1]['w1'], res[1]['b1'], res[1]['w2'], res[1]['b2']]

  def body(agg_ref, m_ref, rbf_ref, dw_ref, db_ref,
           r0w1, r0b1, r0w2, r0b2, r1w1, r1b1, r1w2, r1b2, rwn_ref,
           mn_ref, tn_ref):
    mm = m_ref[...] + _swish(
        jnp.dot(agg_ref[...], dw_ref[...], preferred_element_type=f32)
        + db_ref[...][None, :])
    for (w1, b1, w2, b2) in ((r0w1, r0b1, r0w2, r0b2),
                             (r1w1, r1b1, r1w2, r1b2)):
      h = _swish(jnp.dot(mm, w1[...], preferred_element_type=f32)
                 + b1[...][None, :])
      mm = mm + jnp.dot(h, w2[...], preferred_element_type=f32) + b2[...][None, :]
    mn_ref[...] = mm
    tn_ref[...] = mm * jnp.dot(rbf_ref[...], rwn_ref[...],
                               preferred_element_type=f32)

  return pl.pallas_call(
      body,
      grid=(_NEB2,),
      in_specs=[_mat_spec(_BE2, F), _mat_spec(_BE2, F), _mat_spec(_BE2, R),
                _full_spec((F, F)), _full_spec((F,)),
                _full_spec((F, F)), _full_spec((F,)),
                _full_spec((F, F)), _full_spec((F,)),
                _full_spec((F, F)), _full_spec((F,)),
                _full_spec((F, F)), _full_spec((F,)),
                _full_spec((R, F))],
      out_specs=[_mat_spec(_BE2, F), _mat_spec(_BE2, F)],
      out_shape=[jax.ShapeDtypeStruct((N_EDGES, F), f32),
                 jax.ShapeDtypeStruct((N_EDGES, F), f32)],
  )(agg, m, rbf, dw, db, *rp, rbf_wn)


_BA = 1000
_NAB = N_ATOMS // _BA


def _tc_atom_out(parts, outp):
  ws = []
  for pb in outp:
    ws += [pb['w1'], pb['b1'], pb['w2'], pb['b2']]

  def body(p0_ref, p1_ref, p2_ref,
           w10, b10, w20, b20, w11, b11, w21, b21, w12, b12, w22, b22,
           o_ref):
    @pl.when(pl.program_id(0) == 0)
    def _():
      o_ref[...] = jnp.zeros((1, 1), f32)

    tot = jnp.zeros((), f32)
    for (p_ref, w1, b1, w2, b2) in (
        (p0_ref, w10, b10, w20, b20),
        (p1_ref, w11, b11, w21, b21),
        (p2_ref, w12, b12, w22, b22)):
      af = p_ref[0] + p_ref[1]
      h = _swish(af)
      h2 = _swish(jnp.dot(h, w1[...], preferred_element_type=f32)
                  + b1[...][None, :])
      o = jnp.dot(h2, w2[...], preferred_element_type=f32) + b2[...][None, :]
      tot = tot + jnp.sum(o)
    o_ref[...] = o_ref[...] + tot.reshape(1, 1)

  pspec = pl.BlockSpec((NC, _BA, F), lambda k: (0, k, 0))
  return pl.pallas_call(
      body,
      grid=(_NAB,),
      in_specs=[pspec, pspec, pspec,
                _full_spec((F, F // 2)), _full_spec((F // 2,)),
                _full_spec((F // 2, 1)), _full_spec((1,)),
                _full_spec((F, F // 2)), _full_spec((F // 2,)),
                _full_spec((F // 2, 1)), _full_spec((1,)),
                _full_spec((F, F // 2)), _full_spec((F // 2,)),
                _full_spec((F // 2, 1)), _full_spec((1,))],
      out_specs=_full_spec((1, 1)),
      out_shape=jax.ShapeDtypeStruct((1, 1), f32),
  )(*parts, *ws)


def kernel(z, pos, i, j, idx_kj, idx_ji, params):
  del z
  pxa, pya, pza = pos[:, 0], pos[:, 1], pos[:, 2]

  d2 = _sc_edge_d2(pxa, pya, pza, i, j)
  dot, n1, n2, d2kj = _sc_triplet_geom(pxa, pya, pza, i, j, idx_kj, idx_ji, d2)

  emb = params['emb']
  cvec = (emb['atom'][0] * emb['atom'][0]).astype(f32)
  outp = params['out']
  rbf, m, t0 = _tc_edge(d2, cvec, emb['rbf_w'], emb['rbf_b'], outp[0]['rbf_w'])

  ib0, ib1 = params['inter'][0], params['inter'][1]
  sf0, sf1 = _tc_triplet(dot, n1, n2, d2kj,
                         ib0['sbf_w'], ib0['sbf_b'], ib1['sbf_w'], ib1['sbf_b'])

  parts = [_sc_scatter_atoms(t0, i)]
  sfs = (sf0, sf1)
  for blk, ib in enumerate((ib0, ib1)):
    mkj = _tc_mkj(m, ib['m_w'], ib['m_b'])
    x2 = _sc_row_gather(mkj, idx_kj)
    bmat = jnp.transpose(ib['bil_w'], (2, 1, 0)).reshape(F, 8 * F)
    bil = _tc_bil(x2, sfs[blk], bmat, ib['bil_b'])
    agg = _sc_scatter_edges(bil, idx_ji)
    m, tn = _tc_update(agg, m, rbf, ib['down_w'], ib['down_b'], ib['res'],
                       outp[blk + 1]['rbf_w'])
    parts.append(_sc_scatter_atoms(tn, i))

  total = _tc_atom_out(parts, outp)
  return total[0, 0]


# speedup vs baseline: 1.8947x; 1.0260x over previous
import functools

import jax
import jax.numpy as jnp
import numpy as np
from jax import lax
from jax.experimental import pallas as pl
from jax.experimental.pallas import tpu as pltpu
from jax.experimental.pallas import tpu_sc as plsc

N_ATOMS = 10000
N_EDGES = 160000
N_TRIPLETS = 160000
F = 128
R = 6
S = 7
CUTOFF = 5.0
P = 7

NC, NS, L = 2, 16, 16
NW = NC * NS
EPW = N_EDGES // NW
TPW = N_TRIPLETS // NW
TPS = N_TRIPLETS // NS

def _sc_mesh():
  return plsc.VectorSubcoreMesh(
      core_axis_name="c", subcore_axis_name="s", num_cores=NC, num_subcores=NS)

f32 = jnp.float32
i32 = jnp.int32


def _swish(x):
  return x * jax.nn.sigmoid(x)


def _envelope(x):
  p = P
  a = -(p + 1) * (p + 2) / 2.0
  b = p * (p + 2)
  c = -p * (p + 1) / 2.0
  xp0 = x ** (p - 1)
  xp1 = xp0 * x
  xp2 = xp1 * x
  return 1.0 / x + a * xp0 + b * xp1 + c * xp2


def _wid():
  return lax.axis_index("c") * NS + lax.axis_index("s")


def _sc_edge_d2(pxa, pya, pza, i_arr, j_arr):
  @functools.partial(
      pl.kernel,
      out_type=jax.ShapeDtypeStruct((N_EDGES,), f32),
      mesh=_sc_mesh(),
      compiler_params=pltpu.CompilerParams(needs_layout_passes=False),
      scratch_types=[
          pltpu.VMEM((N_ATOMS,), f32),
          pltpu.VMEM((N_ATOMS,), f32),
          pltpu.VMEM((N_ATOMS,), f32),
          pltpu.VMEM((EPW,), i32),
          pltpu.VMEM((EPW,), i32),
          pltpu.VMEM((EPW,), f32),
      ],
  )
  def k(px_hbm, py_hbm, pz_hbm, i_hbm, j_hbm, d2_hbm, px, py, pz, iv, jv, ov):
    w = _wid()
    base = w * EPW
    pltpu.sync_copy(px_hbm, px)
    pltpu.sync_copy(py_hbm, py)
    pltpu.sync_copy(pz_hbm, pz)
    pltpu.sync_copy(i_hbm.at[pl.ds(base, EPW)], iv)
    pltpu.sync_copy(j_hbm.at[pl.ds(base, EPW)], jv)

    def body(g, _):
      st = jnp.minimum(g * L, EPW - L)
      a16 = iv[pl.ds(st, L)]
      b16 = jv[pl.ds(st, L)]
      dx = plsc.load_gather(px, [a16]) - plsc.load_gather(px, [b16])
      dy = plsc.load_gather(py, [a16]) - plsc.load_gather(py, [b16])
      dz = plsc.load_gather(pz, [a16]) - plsc.load_gather(pz, [b16])
      ov[pl.ds(st, L)] = dx * dx + dy * dy + dz * dz
      return 0

    lax.fori_loop(0, (EPW + L - 1) // L, body, 0)
    pltpu.sync_copy(ov, d2_hbm.at[pl.ds(base, EPW)])

  return k(pxa, pya, pza, i_arr, j_arr)


def _sc_triplet_geom(pxa, pya, pza, i_arr, j_arr, idx_kj, idx_ji, d2):
  out_t = tuple(jax.ShapeDtypeStruct((N_TRIPLETS,), f32) for _ in range(4))

  @functools.partial(
      pl.kernel,
      out_type=out_t,
      mesh=_sc_mesh(),
      compiler_params=pltpu.CompilerParams(needs_layout_passes=False),
      scratch_types=[
          pltpu.VMEM((N_ATOMS,), f32),
          pltpu.VMEM((N_ATOMS,), f32),
          pltpu.VMEM((N_ATOMS,), f32),
          pltpu.VMEM((TPW,), i32),
          pltpu.VMEM((TPW,), i32),
          pltpu.VMEM((TPW,), i32),
          pltpu.VMEM((TPW,), i32),
          pltpu.VMEM((TPW,), i32),
          pltpu.VMEM((TPW,), f32),
          pltpu.VMEM((TPW,), f32),
          pltpu.VMEM((TPW,), f32),
          pltpu.VMEM((TPW,), f32),
          pltpu.SemaphoreType.DMA,
      ],
  )
  def k(px_hbm, py_hbm, pz_hbm, i_hbm, j_hbm, kj_hbm, ji_hbm, d2_hbm,
        dot_hbm, n1_hbm, n2_hbm, d2kj_hbm,
        px, py, pz, jiv, kjv, av, bv, kv, dkv, odot, on1, on2, sem):
    w = _wid()
    base = w * TPW
    pltpu.sync_copy(px_hbm, px)
    pltpu.sync_copy(py_hbm, py)
    pltpu.sync_copy(pz_hbm, pz)
    pltpu.sync_copy(ji_hbm.at[pl.ds(base, TPW)], jiv)
    pltpu.sync_copy(kj_hbm.at[pl.ds(base, TPW)], kjv)
    pltpu.async_copy(i_hbm.at[jiv], av, sem).wait()
    pltpu.async_copy(j_hbm.at[jiv], bv, sem).wait()
    pltpu.async_copy(j_hbm.at[kjv], kv, sem).wait()
    pltpu.async_copy(d2_hbm.at[kjv], dkv, sem).wait()

    def body(g, _):
      st = jnp.minimum(g * L, TPW - L)
      a16 = av[pl.ds(st, L)]
      b16 = bv[pl.ds(st, L)]
      k16 = kv[pl.ds(st, L)]
      bx = plsc.load_gather(px, [b16])
      by = plsc.load_gather(py, [b16])
      bz = plsc.load_gather(pz, [b16])
      v1x = plsc.load_gather(px, [a16]) - bx
      v1y = plsc.load_gather(py, [a16]) - by
      v1z = plsc.load_gather(pz, [a16]) - bz
      v2x = plsc.load_gather(px, [k16]) - bx
      v2y = plsc.load_gather(py, [k16]) - by
      v2z = plsc.load_gather(pz, [k16]) - bz
      odot[pl.ds(st, L)] = v1x * v2x + v1y * v2y + v1z * v2z
      on1[pl.ds(st, L)] = v1x * v1x + v1y * v1y + v1z * v1z
      on2[pl.ds(st, L)] = v2x * v2x + v2y * v2y + v2z * v2z
      return 0

    lax.fori_loop(0, (TPW + L - 1) // L, body, 0)
    pltpu.sync_copy(odot, dot_hbm.at[pl.ds(base, TPW)])
    pltpu.sync_copy(on1, n1_hbm.at[pl.ds(base, TPW)])
    pltpu.sync_copy(on2, n2_hbm.at[pl.ds(base, TPW)])
    pltpu.sync_copy(dkv, d2kj_hbm.at[pl.ds(base, TPW)])

  return k(pxa, pya, pza, i_arr, j_arr, idx_kj, idx_ji, d2)


_G_CHUNK = 200


def _sc_row_gather(tab, idx):
  n_chunks = TPW // _G_CHUNK
  dt = tab.dtype
  dcols = tab.shape[1]

  @functools.partial(
      pl.kernel,
      out_type=jax.ShapeDtypeStruct((N_TRIPLETS, dcols), dt),
      mesh=_sc_mesh(),
      compiler_params=pltpu.CompilerParams(needs_layout_passes=False),
      scratch_types=[
          pltpu.VMEM((_G_CHUNK,), i32),
          pltpu.VMEM((_G_CHUNK, dcols), dt),
          pltpu.SemaphoreType.DMA,
      ],
  )
  def k(tab_hbm, idx_hbm, out_hbm, idxv, rows, sem):
    w = _wid()
    base = w * TPW

    def body(ci, _):
      off = pl.multiple_of(base + ci * _G_CHUNK, 8)
      pltpu.sync_copy(idx_hbm.at[pl.ds(off, _G_CHUNK)], idxv)
      pltpu.async_copy(tab_hbm.at[idxv], rows, sem).wait()
      pltpu.sync_copy(rows, out_hbm.at[pl.ds(off, _G_CHUNK)])
      return 0

    lax.fori_loop(0, n_chunks, body, 0)

  return k(tab, idx)


_CH = 8000
_CHP = 8192
_EPC = N_EDGES // NC
_NCHUNK = _EPC // _CH
_ZROWS = _CHP // NS
_WROWS = 1000


def _sc_scatter_edges(rows_hbm_arr, idx):
  @functools.partial(
      pl.kernel,
      out_type=jax.ShapeDtypeStruct((N_EDGES, F), f32),
      mesh=_sc_mesh(),
      compiler_params=pltpu.CompilerParams(needs_layout_passes=False),
      scratch_types=[
          pltpu.VMEM((TPS,), i32),
          pltpu.VMEM((TPS + 9 * L,), i32),
          pltpu.VMEM((TPS + 9 * L,), i32),
          pltpu.VMEM((4 * L, F), f32),
          pltpu.VMEM((L, F), f32),
          pltpu.VMEM_SHARED((_CHP, F), f32),
          pltpu.SemaphoreType.DMA,
          pltpu.SemaphoreType.DMA,
      ],
  )
  def k(rows_hbm, idx_hbm, agg_hbm, idxv, tlist, llist, rbuf, zrow, acc,
        sem, sem2):
    c = lax.axis_index("c")
    s = lax.axis_index("s")
    sbase = s * TPS
    pltpu.sync_copy(idx_hbm.at[pl.ds(sbase, TPS)], idxv)

    def zfill(r, _):
      for gg in range(F // L):
        zrow[r, pl.ds(gg * L, L)] = jnp.zeros((L,), f32)
      return 0

    lax.fori_loop(0, L, zfill, 0)

    def chunk_body(ci, _):
      cbase = c * _EPC + ci * _CH
      def zcopy(r, _):
        pltpu.sync_copy(
            zrow, acc.at[pl.ds(pl.multiple_of(s * _ZROWS + r * L, 8), L)])
        return 0

      lax.fori_loop(0, _ZROWS // L, zcopy, 0)
      plsc.subcore_barrier()

      def scan_body(g, ptr):
        v = idxv[pl.ds(g * L, L)]
        local = v - cbase
        m = (local >= 0) & (local < _CH)
        tglob = lax.iota(i32, L) + (sbase + g * L)
        plsc.store_compressed(tlist.at[pl.ds(ptr, L)], tglob, mask=m)
        plsc.store_compressed(llist.at[pl.ds(ptr, L)], local, mask=m)
        return ptr + jnp.sum(jnp.where(m, 1, 0).astype(i32))

      ptr = lax.fori_loop(0, TPS // L, scan_body, jnp.int32(0))
      for kk in range(8):
        tlist[pl.ds(ptr + kk * L, L)] = jnp.zeros((L,), i32)
        llist[pl.ds(ptr + kk * L, L)] = jnp.full((L,), _CH, i32)

      nb = (ptr + (L - 1)) // L

      def batch_body(b, _):
        tvec = tlist[pl.ds(b * L, L)]
        lvec = llist[pl.ds(b * L, L)]
        pltpu.async_copy(rows_hbm.at[tvec], rbuf.at[pl.ds(0, L)], sem).wait()
        pltpu.sync_copy(rbuf.at[pl.ds(0, L)], acc.at[lvec], add=True)
        return 0

      lax.fori_loop(0, nb, batch_body, 0)
      plsc.subcore_barrier()
      @pl.when(s < 8)
      def _():
        pltpu.sync_copy(
            acc.at[pl.ds(pl.multiple_of(s * _WROWS, 8), _WROWS)],
            agg_hbm.at[pl.ds(pl.multiple_of(cbase + s * _WROWS, 8), _WROWS)])
      plsc.subcore_barrier()
      return 0

    lax.fori_loop(0, _NCHUNK, chunk_body, 0)

  return k(rows_hbm_arr, idx)


_NP_PAD = 10240
_AZROWS = _NP_PAD // NS


def _sc_scatter_atoms(rows_hbm_arr, i_arr):
  n_full = EPW // L
  tail_valid = EPW - n_full * L

  @functools.partial(
      pl.kernel,
      out_type=jax.ShapeDtypeStruct((NC, _NP_PAD, F), f32),
      mesh=_sc_mesh(),
      compiler_params=pltpu.CompilerParams(needs_layout_passes=False),
      scratch_types=[
          pltpu.VMEM((EPW,), i32),
          pltpu.VMEM((4 * L, F), f32),
          pltpu.VMEM((4 * L, F), f32),
          pltpu.VMEM((L, F), f32),
          pltpu.VMEM_SHARED((_NP_PAD, F), f32),
          pltpu.SemaphoreType.DMA,
          pltpu.SemaphoreType.DMA,
          pltpu.SemaphoreType.DMA,
      ],
  )
  def k(rows_hbm, i_hbm, part_hbm, iv, rbufa, rbufb, zrow, acc,
        sema, semb, sems):
    c = lax.axis_index("c")
    s = lax.axis_index("s")
    w = c * NS + s
    base = w * EPW
    pltpu.sync_copy(i_hbm.at[pl.ds(base, EPW)], iv)

    def zfill(r, _):
      for gg in range(F // L):
        zrow[r, pl.ds(gg * L, L)] = jnp.zeros((L,), f32)
      return 0

    lax.fori_loop(0, L, zfill, 0)

    def zcopy(r, _):
      pltpu.sync_copy(
          zrow, acc.at[pl.ds(pl.multiple_of(s * _AZROWS + r * L, 8), L)])
      return 0

    lax.fori_loop(0, _AZROWS // L, zcopy, 0)
    plsc.subcore_barrier()

    lane = lax.iota(i32, L)
    n_pair = n_full // 8

    def body(m, _):
      st_a = m * (8 * L)
      st_b = st_a + 4 * L
      lda = pltpu.async_copy(
          rows_hbm.at[pl.ds(pl.multiple_of(base + st_a, 8), 4 * L)],
          rbufa, sema)
      ldb = pltpu.async_copy(
          rows_hbm.at[pl.ds(pl.multiple_of(base + st_b, 8), 4 * L)],
          rbufb, semb)
      sds = []
      lda.wait()
      for kk in range(4):
        ivec = iv[pl.ds(st_a + kk * L, L)]
        sds.append(pltpu.async_copy(
            rbufa.at[pl.ds(kk * L, L)], acc.at[ivec], sems, add=True))
      ldb.wait()
      for kk in range(4):
        ivec = iv[pl.ds(st_b + kk * L, L)]
        sds.append(pltpu.async_copy(
            rbufb.at[pl.ds(kk * L, L)], acc.at[ivec], sems, add=True))
      for d in sds:
        d.wait()
      return 0

    lax.fori_loop(0, n_pair, body, 0)
    for g in range(n_pair * 8, n_full):
      st = g * L
      pltpu.sync_copy(
          rows_hbm.at[pl.ds(pl.multiple_of(base + st, 8), L)],
          rbufa.at[pl.ds(0, L)])
      ivec = iv[pl.ds(st, L)]
      pltpu.sync_copy(rbufa.at[pl.ds(0, L)], acc.at[ivec], add=True)
    if tail_valid:
      st = EPW - L
      pltpu.sync_copy(rows_hbm.at[pl.ds(pl.multiple_of(base + st, 8), L)],
                      rbufa.at[pl.ds(0, L)])
      ivec = iv[pl.ds(st, L)]
      ivec = jnp.where(lane < (L - tail_valid), jnp.full((L,), N_ATOMS, i32),
                       ivec)
      pltpu.sync_copy(rbufa.at[pl.ds(0, L)], acc.at[ivec], add=True)

    plsc.subcore_barrier()
    pltpu.sync_copy(
        acc.at[pl.ds(pl.multiple_of(s * _AZROWS, 8), _AZROWS)],
        part_hbm.at[c, pl.ds(pl.multiple_of(s * _AZROWS, 8), _AZROWS)])

  return k(rows_hbm_arr, i_arr)


_BE = 256
_NEB = N_EDGES // _BE
_BE2 = 2000
_NEB2 = N_EDGES // _BE2

def _freq_row():
  return (lax.broadcasted_iota(jnp.int32, (1, R), 1).astype(f32) + 1.0) * np.pi


def _sbf_consts():
  q = lax.broadcasted_iota(jnp.int32, (1, S * R), 1)
  r = (q % R).astype(f32)
  s = (q // R).astype(f32)
  bwf = (r + 1.0) * np.pi + s * (S - 1) * np.pi
  return bwf, s


def _vec_spec(b):
  return pl.BlockSpec((b,), lambda k: (k,))


def _mat_spec(b, d):
  return pl.BlockSpec((b, d), lambda k: (k, 0))


def _full_spec(shape):
  nd = len(shape)
  return pl.BlockSpec(shape, lambda k: (0,) * nd)


def _tc_edge(d2, cvec, emb_w, emb_b, rbf_w0, mw0, mb0):
  def body(d2_ref, c_ref, ew_ref, eb_ref, rw0_ref, mw_ref, mb_ref,
           rbf_ref, m0_ref, t0_ref, mkj_ref):
    d2t = d2_ref[...]
    dist = jnp.sqrt(d2t)
    ds_ = dist / CUTOFF
    env = _envelope(ds_)
    rbf = env[:, None] * jnp.sin(_freq_row() * ds_[:, None]) / dist[:, None]
    rbf_ref[...] = rbf
    m0 = c_ref[...][None, :] * _swish(
        jnp.dot(rbf, ew_ref[...], preferred_element_type=f32)
        + eb_ref[...][None, :])
    m0_ref[...] = m0
    t0_ref[...] = m0 * jnp.dot(rbf, rw0_ref[...], preferred_element_type=f32)
    mkj_ref[...] = _swish(
        jnp.dot(m0, mw_ref[...], preferred_element_type=f32)
        + mb_ref[...][None, :])

  return pl.pallas_call(
      body,
      grid=(_NEB,),
      in_specs=[_vec_spec(_BE), _full_spec((F,)), _full_spec((R, F)),
                _full_spec((F,)), _full_spec((R, F)),
                _full_spec((F, F)), _full_spec((F,))],
      out_specs=[_mat_spec(_BE, R), _mat_spec(_BE, F), _mat_spec(_BE, F),
                 _mat_spec(_BE, F)],
      out_shape=[jax.ShapeDtypeStruct((N_EDGES, R), f32),
                 jax.ShapeDtypeStruct((N_EDGES, F), f32),
                 jax.ShapeDtypeStruct((N_EDGES, F), f32),
                 jax.ShapeDtypeStruct((N_EDGES, F), f32)],
  )(d2, cvec, emb_w, emb_b, rbf_w0, mw0, mb0)


def _tc_triplet(dot, n1, n2, d2kj, sw0, sb0, sw1, sb1):
  def body(dot_ref, n1_ref, n2_ref, dk_ref, w0_ref, b0_ref, w1_ref, b1_ref,
           sf0_ref, sf1_ref):
    dott = dot_ref[...]
    nrm = jnp.sqrt(n1_ref[...] + 1e-12) * jnp.sqrt(n2_ref[...] + 1e-12)
    ca = jnp.clip(dott / (nrm + 1e-7), -1.0 + 1e-6, 1.0 - 1e-6)
    dk = jnp.sqrt(dk_ref[...]) / CUTOFF
    env = _envelope(dk)
    bwf, shwf = _sbf_consts()
    cprev = jnp.ones_like(ca)
    ccur = ca
    sph = (shwf == 0.0).astype(f32) * cprev[:, None]
    sph = sph + (shwf == 1.0).astype(f32) * ccur[:, None]
    for s_ in range(2, S):
      cnext = 2.0 * ca * ccur - cprev
      cprev, ccur = ccur, cnext
      sph = sph + (shwf == float(s_)).astype(f32) * ccur[:, None]
    sbf = env[:, None] * jnp.sin(bwf * dk[:, None]) * sph
    sf0_ref[...] = _swish(
        jnp.dot(sbf, w0_ref[...], preferred_element_type=f32)
        + b0_ref[...][None, :])
    sf1_ref[...] = _swish(
        jnp.dot(sbf, w1_ref[...], preferred_element_type=f32)
        + b1_ref[...][None, :])

  return pl.pallas_call(
      body,
      grid=(_NEB,),
      in_specs=[_vec_spec(_BE)] * 4
      + [_full_spec((S * R, 8)), _full_spec((8,)),
         _full_spec((S * R, 8)), _full_spec((8,))],
      out_specs=[_mat_spec(_BE, 8), _mat_spec(_BE, 8)],
      out_shape=[jax.ShapeDtypeStruct((N_TRIPLETS, 8), f32),
                 jax.ShapeDtypeStruct((N_TRIPLETS, 8), f32)],
  )(dot, n1, n2, d2kj, sw0, sb0, sw1, sb1)


def _tc_mkj(m, mw, mb):
  def body(m_ref, w_ref, b_ref, o_ref):
    o_ref[...] = _swish(
        jnp.dot(m_ref[...], w_ref[...], preferred_element_type=f32)
        + b_ref[...][None, :])

  return pl.pallas_call(
      body,
      grid=(_NEB2,),
      in_specs=[_mat_spec(_BE2, F), _full_spec((F, F)), _full_spec((F,))],
      out_specs=_mat_spec(_BE2, F),
      out_shape=jax.ShapeDtypeStruct((N_EDGES, F), f32),
  )(m, mw, mb)


def _tc_bil(x2, sf, bmat, bil_b):
  def body(x_ref, sf_ref, bm_ref, bb_ref, o_ref):
    y = jnp.dot(x_ref[...].astype(jnp.bfloat16), bm_ref[...],
                preferred_element_type=f32)
    sft = sf_ref[...]
    acc = bb_ref[...][None, :]
    for ii in range(8):
      acc = acc + sft[:, ii][:, None] * y[:, ii * F:(ii + 1) * F]
    o_ref[...] = acc

  return pl.pallas_call(
      body,
      grid=(_NEB2,),
      in_specs=[_mat_spec(_BE2, F), _mat_spec(_BE2, 8),
                _full_spec((F, 8 * F)), _full_spec((F,))],
      out_specs=_mat_spec(_BE2, F),
      out_shape=jax.ShapeDtypeStruct((N_TRIPLETS, F), f32),
  )(x2, sf, bmat.astype(jnp.bfloat16), bil_b)


def _tc_update(agg, m, rbf, dw, db, res, rbf_wn, mw_next=None, mb_next=None):
  rp = [res[0]['w1'], res[0]['b1'], res[0]['w2'], res[0]['b2'],
        res[1]['w1'], res[1]['b1'], res[1]['w2'], res[1]['b2']]
  emit_mkj = mw_next is not None

  def body(*refs):
    if emit_mkj:
      (agg_ref, m_ref, rbf_ref, dw_ref, db_ref,
       r0w1, r0b1, r0w2, r0b2, r1w1, r1b1, r1w2, r1b2, rwn_ref,
       mwn_ref, mbn_ref, mn_ref, tn_ref, mkj_ref) = refs
    else:
      (agg_ref, m_ref, rbf_ref, dw_ref, db_ref,
       r0w1, r0b1, r0w2, r0b2, r1w1, r1b1, r1w2, r1b2, rwn_ref,
       mn_ref, tn_ref) = refs
    mm = m_ref[...] + _swish(
        jnp.dot(agg_ref[...], dw_ref[...], preferred_element_type=f32)
        + db_ref[...][None, :])
    for (w1, b1, w2, b2) in ((r0w1, r0b1, r0w2, r0b2),
                             (r1w1, r1b1, r1w2, r1b2)):
      h = _swish(jnp.dot(mm, w1[...], preferred_element_type=f32)
                 + b1[...][None, :])
      mm = mm + jnp.dot(h, w2[...], preferred_element_type=f32) + b2[...][None, :]
    mn_ref[...] = mm
    tn_ref[...] = mm * jnp.dot(rbf_ref[...], rwn_ref[...],
                               preferred_element_type=f32)
    if emit_mkj:
      mkj_ref[...] = _swish(
          jnp.dot(mm, mwn_ref[...], preferred_element_type=f32)
          + mbn_ref[...][None, :])

  ins = [_mat_spec(_BE2, F), _mat_spec(_BE2, F), _mat_spec(_BE2, R),
         _full_spec((F, F)), _full_spec((F,)),
         _full_spec((F, F)), _full_spec((F,)),
         _full_spec((F, F)), _full_spec((F,)),
         _full_spec((F, F)), _full_spec((F,)),
         _full_spec((F, F)), _full_spec((F,)),
         _full_spec((R, F))]
  outs = [_mat_spec(_BE2, F), _mat_spec(_BE2, F)]
  oshapes = [jax.ShapeDtypeStruct((N_EDGES, F), f32),
             jax.ShapeDtypeStruct((N_EDGES, F), f32)]
  args = [agg, m, rbf, dw, db] + rp + [rbf_wn]
  if emit_mkj:
    ins += [_full_spec((F, F)), _full_spec((F,))]
    outs.append(_mat_spec(_BE2, F))
    oshapes.append(jax.ShapeDtypeStruct((N_EDGES, F), f32))
    args += [mw_next, mb_next]

  return pl.pallas_call(
      body,
      grid=(_NEB2,),
      in_specs=ins,
      out_specs=outs,
      out_shape=oshapes,
  )(*args)


_BA = 1000
_NAB = N_ATOMS // _BA


def _tc_atom_out(parts, outp):
  ws = []
  for pb in outp:
    ws += [pb['w1'], pb['b1'], pb['w2'], pb['b2']]

  def body(p0_ref, p1_ref, p2_ref,
           w10, b10, w20, b20, w11, b11, w21, b21, w12, b12, w22, b22,
           o_ref):
    @pl.when(pl.program_id(0) == 0)
    def _():
      o_ref[...] = jnp.zeros((1, 1), f32)

    tot = jnp.zeros((), f32)
    for (p_ref, w1, b1, w2, b2) in (
        (p0_ref, w10, b10, w20, b20),
        (p1_ref, w11, b11, w21, b21),
        (p2_ref, w12, b12, w22, b22)):
      af = p_ref[0] + p_ref[1]
      h = _swish(af)
      h2 = _swish(jnp.dot(h, w1[...], preferred_element_type=f32)
                  + b1[...][None, :])
      o = jnp.dot(h2, w2[...], preferred_element_type=f32) + b2[...][None, :]
      tot = tot + jnp.sum(o)
    o_ref[...] = o_ref[...] + tot.reshape(1, 1)

  pspec = pl.BlockSpec((NC, _BA, F), lambda k: (0, k, 0))
  return pl.pallas_call(
      body,
      grid=(_NAB,),
      in_specs=[pspec, pspec, pspec,
                _full_spec((F, F // 2)), _full_spec((F // 2,)),
                _full_spec((F // 2, 1)), _full_spec((1,)),
                _full_spec((F, F // 2)), _full_spec((F // 2,)),
                _full_spec((F // 2, 1)), _full_spec((1,)),
                _full_spec((F, F // 2)), _full_spec((F // 2,)),
                _full_spec((F // 2, 1)), _full_spec((1,))],
      out_specs=_full_spec((1, 1)),
      out_shape=jax.ShapeDtypeStruct((1, 1), f32),
  )(*parts, *ws)


def kernel(z, pos, i, j, idx_kj, idx_ji, params):
  del z
  pxa, pya, pza = pos[:, 0], pos[:, 1], pos[:, 2]

  d2 = _sc_edge_d2(pxa, pya, pza, i, j)
  dot, n1, n2, d2kj = _sc_triplet_geom(pxa, pya, pza, i, j, idx_kj, idx_ji, d2)

  emb = params['emb']
  cvec = (emb['atom'][0] * emb['atom'][0]).astype(f32)
  outp = params['out']
  ib0_mw = params['inter'][0]['m_w']
  ib0_mb = params['inter'][0]['m_b']
  rbf, m, t0, mkj = _tc_edge(d2, cvec, emb['rbf_w'], emb['rbf_b'],
                             outp[0]['rbf_w'], ib0_mw, ib0_mb)

  ib0, ib1 = params['inter'][0], params['inter'][1]
  sf0, sf1 = _tc_triplet(dot, n1, n2, d2kj,
                         ib0['sbf_w'], ib0['sbf_b'], ib1['sbf_w'], ib1['sbf_b'])

  parts = [_sc_scatter_atoms(t0, i)]
  sfs = (sf0, sf1)
  for blk, ib in enumerate((ib0, ib1)):
    x2 = _sc_row_gather(mkj, idx_kj)
    bmat = jnp.transpose(ib['bil_w'], (2, 1, 0)).reshape(F, 8 * F)
    bil = _tc_bil(x2, sfs[blk], bmat, ib['bil_b'])
    agg = _sc_scatter_edges(bil, idx_ji)
    if blk == 0:
      m, tn, mkj = _tc_update(agg, m, rbf, ib['down_w'], ib['down_b'],
                              ib['res'], outp[blk + 1]['rbf_w'],
                              ib1['m_w'], ib1['m_b'])
    else:
      m, tn = _tc_update(agg, m, rbf, ib['down_w'], ib['down_b'],
                         ib['res'], outp[blk + 1]['rbf_w'])
    parts.append(_sc_scatter_atoms(tn, i))

  total = _tc_atom_out(parts, outp)
  return total[0, 0]


# speedup vs baseline: 2.0428x; 1.0782x over previous
import functools

import jax
import jax.numpy as jnp
import numpy as np
from jax import lax
from jax.experimental import pallas as pl
from jax.experimental.pallas import tpu as pltpu
from jax.experimental.pallas import tpu_sc as plsc

N_ATOMS = 10000
N_EDGES = 160000
N_TRIPLETS = 160000
F = 128
R = 6
S = 7
CUTOFF = 5.0
P = 7

NC, NS, L = 2, 16, 16
NW = NC * NS
EPW = N_EDGES // NW
TPW = N_TRIPLETS // NW
TPS = N_TRIPLETS // NS

def _sc_mesh():
  return plsc.VectorSubcoreMesh(
      core_axis_name="c", subcore_axis_name="s", num_cores=NC, num_subcores=NS)

f32 = jnp.float32
i32 = jnp.int32


def _swish(x):
  return x * jax.nn.sigmoid(x)


def _envelope(x):
  p = P
  a = -(p + 1) * (p + 2) / 2.0
  b = p * (p + 2)
  c = -p * (p + 1) / 2.0
  xp0 = x ** (p - 1)
  xp1 = xp0 * x
  xp2 = xp1 * x
  return 1.0 / x + a * xp0 + b * xp1 + c * xp2


def _wid():
  return lax.axis_index("c") * NS + lax.axis_index("s")


def _sc_edge_d2(pxa, pya, pza, i_arr, j_arr):
  @functools.partial(
      pl.kernel,
      out_type=jax.ShapeDtypeStruct((N_EDGES,), f32),
      mesh=_sc_mesh(),
      compiler_params=pltpu.CompilerParams(needs_layout_passes=False),
      scratch_types=[
          pltpu.VMEM((N_ATOMS,), f32),
          pltpu.VMEM((N_ATOMS,), f32),
          pltpu.VMEM((N_ATOMS,), f32),
          pltpu.VMEM((EPW,), i32),
          pltpu.VMEM((EPW,), i32),
          pltpu.VMEM((EPW,), f32),
      ],
  )
  def k(px_hbm, py_hbm, pz_hbm, i_hbm, j_hbm, d2_hbm, px, py, pz, iv, jv, ov):
    w = _wid()
    base = w * EPW
    pltpu.sync_copy(px_hbm, px)
    pltpu.sync_copy(py_hbm, py)
    pltpu.sync_copy(pz_hbm, pz)
    pltpu.sync_copy(i_hbm.at[pl.ds(base, EPW)], iv)
    pltpu.sync_copy(j_hbm.at[pl.ds(base, EPW)], jv)

    def body(g, _):
      st = jnp.minimum(g * L, EPW - L)
      a16 = iv[pl.ds(st, L)]
      b16 = jv[pl.ds(st, L)]
      dx = plsc.load_gather(px, [a16]) - plsc.load_gather(px, [b16])
      dy = plsc.load_gather(py, [a16]) - plsc.load_gather(py, [b16])
      dz = plsc.load_gather(pz, [a16]) - plsc.load_gather(pz, [b16])
      ov[pl.ds(st, L)] = dx * dx + dy * dy + dz * dz
      return 0

    lax.fori_loop(0, (EPW + L - 1) // L, body, 0)
    pltpu.sync_copy(ov, d2_hbm.at[pl.ds(base, EPW)])

  return k(pxa, pya, pza, i_arr, j_arr)


def _sc_triplet_geom(pxa, pya, pza, i_arr, j_arr, idx_kj, idx_ji, d2):
  out_t = tuple(jax.ShapeDtypeStruct((N_TRIPLETS,), f32) for _ in range(4))

  @functools.partial(
      pl.kernel,
      out_type=out_t,
      mesh=_sc_mesh(),
      compiler_params=pltpu.CompilerParams(needs_layout_passes=False),
      scratch_types=[
          pltpu.VMEM((N_ATOMS,), f32),
          pltpu.VMEM((N_ATOMS,), f32),
          pltpu.VMEM((N_ATOMS,), f32),
          pltpu.VMEM((TPW,), i32),
          pltpu.VMEM((TPW,), i32),
          pltpu.VMEM((TPW,), i32),
          pltpu.VMEM((TPW,), i32),
          pltpu.VMEM((TPW,), i32),
          pltpu.VMEM((TPW,), f32),
          pltpu.VMEM((TPW,), f32),
          pltpu.VMEM((TPW,), f32),
          pltpu.VMEM((TPW,), f32),
          pltpu.SemaphoreType.DMA,
      ],
  )
  def k(px_hbm, py_hbm, pz_hbm, i_hbm, j_hbm, kj_hbm, ji_hbm, d2_hbm,
        dot_hbm, n1_hbm, n2_hbm, d2kj_hbm,
        px, py, pz, jiv, kjv, av, bv, kv, dkv, odot, on1, on2, sem):
    w = _wid()
    base = w * TPW
    pltpu.sync_copy(px_hbm, px)
    pltpu.sync_copy(py_hbm, py)
    pltpu.sync_copy(pz_hbm, pz)
    pltpu.sync_copy(ji_hbm.at[pl.ds(base, TPW)], jiv)
    pltpu.sync_copy(kj_hbm.at[pl.ds(base, TPW)], kjv)
    pltpu.async_copy(i_hbm.at[jiv], av, sem).wait()
    pltpu.async_copy(j_hbm.at[jiv], bv, sem).wait()
    pltpu.async_copy(j_hbm.at[kjv], kv, sem).wait()
    pltpu.async_copy(d2_hbm.at[kjv], dkv, sem).wait()

    def body(g, _):
      st = jnp.minimum(g * L, TPW - L)
      a16 = av[pl.ds(st, L)]
      b16 = bv[pl.ds(st, L)]
      k16 = kv[pl.ds(st, L)]
      bx = plsc.load_gather(px, [b16])
      by = plsc.load_gather(py, [b16])
      bz = plsc.load_gather(pz, [b16])
      v1x = plsc.load_gather(px, [a16]) - bx
      v1y = plsc.load_gather(py, [a16]) - by
      v1z = plsc.load_gather(pz, [a16]) - bz
      v2x = plsc.load_gather(px, [k16]) - bx
      v2y = plsc.load_gather(py, [k16]) - by
      v2z = plsc.load_gather(pz, [k16]) - bz
      odot[pl.ds(st, L)] = v1x * v2x + v1y * v2y + v1z * v2z
      on1[pl.ds(st, L)] = v1x * v1x + v1y * v1y + v1z * v1z
      on2[pl.ds(st, L)] = v2x * v2x + v2y * v2y + v2z * v2z
      return 0

    lax.fori_loop(0, (TPW + L - 1) // L, body, 0)
    pltpu.sync_copy(odot, dot_hbm.at[pl.ds(base, TPW)])
    pltpu.sync_copy(on1, n1_hbm.at[pl.ds(base, TPW)])
    pltpu.sync_copy(on2, n2_hbm.at[pl.ds(base, TPW)])
    pltpu.sync_copy(dkv, d2kj_hbm.at[pl.ds(base, TPW)])

  return k(pxa, pya, pza, i_arr, j_arr, idx_kj, idx_ji, d2)


_G_CHUNK = 200


def _sc_row_gather(tab, idx):
  n_chunks = TPW // _G_CHUNK
  dt = tab.dtype
  dcols = tab.shape[1]

  @functools.partial(
      pl.kernel,
      out_type=jax.ShapeDtypeStruct((N_TRIPLETS, dcols), dt),
      mesh=_sc_mesh(),
      compiler_params=pltpu.CompilerParams(needs_layout_passes=False),
      scratch_types=[
          pltpu.VMEM((_G_CHUNK,), i32),
          pltpu.VMEM((_G_CHUNK, dcols), dt),
          pltpu.SemaphoreType.DMA,
      ],
  )
  def k(tab_hbm, idx_hbm, out_hbm, idxv, rows, sem):
    w = _wid()
    base = w * TPW

    def body(ci, _):
      off = pl.multiple_of(base + ci * _G_CHUNK, 8)
      pltpu.sync_copy(idx_hbm.at[pl.ds(off, _G_CHUNK)], idxv)
      pltpu.async_copy(tab_hbm.at[idxv], rows, sem).wait()
      pltpu.sync_copy(rows, out_hbm.at[pl.ds(off, _G_CHUNK)])
      return 0

    lax.fori_loop(0, n_chunks, body, 0)

  return k(tab, idx)


_CH = 8000
_CHP = 8192
_EPC = N_EDGES // NC
_NCHUNK = _EPC // _CH
_ZROWS = _CHP // NS
_WROWS = 1000


def _sc_scatter_edges(rows_hbm_arr, idx):
  @functools.partial(
      pl.kernel,
      out_type=jax.ShapeDtypeStruct((N_EDGES, F), f32),
      mesh=_sc_mesh(),
      compiler_params=pltpu.CompilerParams(needs_layout_passes=False),
      scratch_types=[
          pltpu.VMEM((TPS,), i32),
          pltpu.VMEM((TPS + 9 * L,), i32),
          pltpu.VMEM((TPS + 9 * L,), i32),
          pltpu.VMEM((4 * L, F), f32),
          pltpu.VMEM((L, F), f32),
          pltpu.VMEM_SHARED((_CHP, F), f32),
          pltpu.SemaphoreType.DMA,
          pltpu.SemaphoreType.DMA,
      ],
  )
  def k(rows_hbm, idx_hbm, agg_hbm, idxv, tlist, llist, rbuf, zrow, acc,
        sem, sem2):
    c = lax.axis_index("c")
    s = lax.axis_index("s")
    sbase = s * TPS
    pltpu.sync_copy(idx_hbm.at[pl.ds(sbase, TPS)], idxv)

    def zfill(r, _):
      for gg in range(F // L):
        zrow[r, pl.ds(gg * L, L)] = jnp.zeros((L,), f32)
      return 0

    lax.fori_loop(0, L, zfill, 0)

    def chunk_body(ci, _):
      cbase = c * _EPC + ci * _CH
      def zcopy(r, _):
        pltpu.sync_copy(
            zrow, acc.at[pl.ds(pl.multiple_of(s * _ZROWS + r * L, 8), L)])
        return 0

      lax.fori_loop(0, _ZROWS // L, zcopy, 0)
      plsc.subcore_barrier()

      def scan_body(g, ptr):
        v = idxv[pl.ds(g * L, L)]
        local = v - cbase
        m = (local >= 0) & (local < _CH)
        tglob = lax.iota(i32, L) + (sbase + g * L)
        plsc.store_compressed(tlist.at[pl.ds(ptr, L)], tglob, mask=m)
        plsc.store_compressed(llist.at[pl.ds(ptr, L)], local, mask=m)
        return ptr + jnp.sum(jnp.where(m, 1, 0).astype(i32))

      ptr = lax.fori_loop(0, TPS // L, scan_body, jnp.int32(0))
      for kk in range(8):
        tlist[pl.ds(ptr + kk * L, L)] = jnp.zeros((L,), i32)
        llist[pl.ds(ptr + kk * L, L)] = jnp.full((L,), _CH, i32)

      nb = (ptr + (L - 1)) // L

      def batch_body(b, _):
        tvec = tlist[pl.ds(b * L, L)]
        lvec = llist[pl.ds(b * L, L)]
        pltpu.async_copy(rows_hbm.at[tvec], rbuf.at[pl.ds(0, L)], sem).wait()
        pltpu.sync_copy(rbuf.at[pl.ds(0, L)], acc.at[lvec], add=True)
        return 0

      lax.fori_loop(0, nb, batch_body, 0)
      plsc.subcore_barrier()
      @pl.when(s < 8)
      def _():
        pltpu.sync_copy(
            acc.at[pl.ds(pl.multiple_of(s * _WROWS, 8), _WROWS)],
            agg_hbm.at[pl.ds(pl.multiple_of(cbase + s * _WROWS, 8), _WROWS)])
      plsc.subcore_barrier()
      return 0

    lax.fori_loop(0, _NCHUNK, chunk_body, 0)

  return k(rows_hbm_arr, idx)


_NP_PAD = 10240
_AZROWS = _NP_PAD // NS


def _sc_scatter_atoms(rows_hbm_arr, i_arr):
  n_full = EPW // L
  tail_valid = EPW - n_full * L

  @functools.partial(
      pl.kernel,
      out_type=jax.ShapeDtypeStruct((NC, _NP_PAD, F), f32),
      mesh=_sc_mesh(),
      compiler_params=pltpu.CompilerParams(needs_layout_passes=False),
      scratch_types=[
          pltpu.VMEM((EPW,), i32),
          pltpu.VMEM((4 * L, F), f32),
          pltpu.VMEM((4 * L, F), f32),
          pltpu.VMEM((L, F), f32),
          pltpu.VMEM_SHARED((_NP_PAD, F), f32),
          pltpu.SemaphoreType.DMA,
          pltpu.SemaphoreType.DMA,
          pltpu.SemaphoreType.DMA,
      ],
  )
  def k(rows_hbm, i_hbm, part_hbm, iv, rbufa, rbufb, zrow, acc,
        sema, semb, sems):
    c = lax.axis_index("c")
    s = lax.axis_index("s")
    w = c * NS + s
    base = w * EPW
    pltpu.sync_copy(i_hbm.at[pl.ds(base, EPW)], iv)

    def zfill(r, _):
      for gg in range(F // L):
        zrow[r, pl.ds(gg * L, L)] = jnp.zeros((L,), f32)
      return 0

    lax.fori_loop(0, L, zfill, 0)

    def zcopy(r, _):
      pltpu.sync_copy(
          zrow, acc.at[pl.ds(pl.multiple_of(s * _AZROWS + r * L, 8), L)])
      return 0

    lax.fori_loop(0, _AZROWS // L, zcopy, 0)
    plsc.subcore_barrier()

    lane = lax.iota(i32, L)
    n_pair = n_full // 8

    def body(m, _):
      st_a = m * (8 * L)
      st_b = st_a + 4 * L
      lda = pltpu.async_copy(
          rows_hbm.at[pl.ds(pl.multiple_of(base + st_a, 8), 4 * L)],
          rbufa, sema)
      ldb = pltpu.async_copy(
          rows_hbm.at[pl.ds(pl.multiple_of(base + st_b, 8), 4 * L)],
          rbufb, semb)
      sds = []
      lda.wait()
      for kk in range(4):
        ivec = iv[pl.ds(st_a + kk * L, L)]
        sds.append(pltpu.async_copy(
            rbufa.at[pl.ds(kk * L, L)], acc.at[ivec], sems, add=True))
      ldb.wait()
      for kk in range(4):
        ivec = iv[pl.ds(st_b + kk * L, L)]
        sds.append(pltpu.async_copy(
            rbufb.at[pl.ds(kk * L, L)], acc.at[ivec], sems, add=True))
      for d in sds:
        d.wait()
      return 0

    lax.fori_loop(0, n_pair, body, 0)
    for g in range(n_pair * 8, n_full):
      st = g * L
      pltpu.sync_copy(
          rows_hbm.at[pl.ds(pl.multiple_of(base + st, 8), L)],
          rbufa.at[pl.ds(0, L)])
      ivec = iv[pl.ds(st, L)]
      pltpu.sync_copy(rbufa.at[pl.ds(0, L)], acc.at[ivec], add=True)
    if tail_valid:
      st = EPW - L
      pltpu.sync_copy(rows_hbm.at[pl.ds(pl.multiple_of(base + st, 8), L)],
                      rbufa.at[pl.ds(0, L)])
      ivec = iv[pl.ds(st, L)]
      ivec = jnp.where(lane < (L - tail_valid), jnp.full((L,), N_ATOMS, i32),
                       ivec)
      pltpu.sync_copy(rbufa.at[pl.ds(0, L)], acc.at[ivec], add=True)

    plsc.subcore_barrier()
    pltpu.sync_copy(
        acc.at[pl.ds(pl.multiple_of(s * _AZROWS, 8), _AZROWS)],
        part_hbm.at[c, pl.ds(pl.multiple_of(s * _AZROWS, 8), _AZROWS)])

  return k(rows_hbm_arr, i_arr)


_BE = 1024
_NEB = (N_EDGES + _BE - 1) // _BE
_BE2 = 2000
_NEB2 = N_EDGES // _BE2

def _freq_row():
  return (lax.broadcasted_iota(jnp.int32, (1, R), 1).astype(f32) + 1.0) * np.pi


def _sbf_consts():
  q = lax.broadcasted_iota(jnp.int32, (1, S * R), 1)
  r = (q % R).astype(f32)
  s = (q // R).astype(f32)
  bwf = (r + 1.0) * np.pi + s * (S - 1) * np.pi
  return bwf, s


def _vec_spec(b):
  return pl.BlockSpec((b,), lambda k: (k,))


def _mat_spec(b, d):
  return pl.BlockSpec((b, d), lambda k: (k, 0))


def _full_spec(shape):
  nd = len(shape)
  return pl.BlockSpec(shape, lambda k: (0,) * nd)


def _tc_edge(d2, cvec, emb_w, emb_b, rbf_w0, mw0, mb0):
  def body(d2_ref, c_ref, ew_ref, eb_ref, rw0_ref, mw_ref, mb_ref,
           rbf_ref, m0_ref, t0_ref, mkj_ref):
    d2t = d2_ref[...]
    dist = jnp.sqrt(d2t)
    ds_ = dist / CUTOFF
    env = _envelope(ds_)
    rbf = env[:, None] * jnp.sin(_freq_row() * ds_[:, None]) / dist[:, None]
    rbf_ref[...] = rbf
    m0 = c_ref[...][None, :] * _swish(
        jnp.dot(rbf, ew_ref[...], preferred_element_type=f32)
        + eb_ref[...][None, :])
    m0_ref[...] = m0
    t0_ref[...] = m0 * jnp.dot(rbf, rw0_ref[...], preferred_element_type=f32)
    mkj_ref[...] = _swish(
        jnp.dot(m0, mw_ref[...], preferred_element_type=f32)
        + mb_ref[...][None, :])

  return pl.pallas_call(
      body,
      grid=(_NEB,),
      in_specs=[_vec_spec(_BE), _full_spec((F,)), _full_spec((R, F)),
                _full_spec((F,)), _full_spec((R, F)),
                _full_spec((F, F)), _full_spec((F,))],
      out_specs=[_mat_spec(_BE, R), _mat_spec(_BE, F), _mat_spec(_BE, F),
                 _mat_spec(_BE, F)],
      out_shape=[jax.ShapeDtypeStruct((N_EDGES, R), f32),
                 jax.ShapeDtypeStruct((N_EDGES, F), f32),
                 jax.ShapeDtypeStruct((N_EDGES, F), f32),
                 jax.ShapeDtypeStruct((N_EDGES, F), f32)],
  )(d2, cvec, emb_w, emb_b, rbf_w0, mw0, mb0)


def _tc_triplet(dot, n1, n2, d2kj, sw0, sb0, sw1, sb1):
  def body(dot_ref, n1_ref, n2_ref, dk_ref, w0_ref, b0_ref, w1_ref, b1_ref,
           sf0_ref, sf1_ref):
    dott = dot_ref[...]
    nrm = jnp.sqrt(n1_ref[...] + 1e-12) * jnp.sqrt(n2_ref[...] + 1e-12)
    ca = jnp.clip(dott / (nrm + 1e-7), -1.0 + 1e-6, 1.0 - 1e-6)
    dk = jnp.sqrt(dk_ref[...]) / CUTOFF
    env = _envelope(dk)
    bwf, shwf = _sbf_consts()
    cprev = jnp.ones_like(ca)
    ccur = ca
    sph = (shwf == 0.0).astype(f32) * cprev[:, None]
    sph = sph + (shwf == 1.0).astype(f32) * ccur[:, None]
    for s_ in range(2, S):
      cnext = 2.0 * ca * ccur - cprev
      cprev, ccur = ccur, cnext
      sph = sph + (shwf == float(s_)).astype(f32) * ccur[:, None]
    sbf = env[:, None] * jnp.sin(bwf * dk[:, None]) * sph
    sf0_ref[...] = _swish(
        jnp.dot(sbf, w0_ref[...], preferred_element_type=f32)
        + b0_ref[...][None, :])
    sf1_ref[...] = _swish(
        jnp.dot(sbf, w1_ref[...], preferred_element_type=f32)
        + b1_ref[...][None, :])

  return pl.pallas_call(
      body,
      grid=(_NEB,),
      in_specs=[_vec_spec(_BE)] * 4
      + [_full_spec((S * R, 8)), _full_spec((8,)),
         _full_spec((S * R, 8)), _full_spec((8,))],
      out_specs=[_mat_spec(_BE, 8), _mat_spec(_BE, 8)],
      out_shape=[jax.ShapeDtypeStruct((N_TRIPLETS, 8), f32),
                 jax.ShapeDtypeStruct((N_TRIPLETS, 8), f32)],
  )(dot, n1, n2, d2kj, sw0, sb0, sw1, sb1)


def _tc_mkj(m, mw, mb):
  def body(m_ref, w_ref, b_ref, o_ref):
    o_ref[...] = _swish(
        jnp.dot(m_ref[...], w_ref[...], preferred_element_type=f32)
        + b_ref[...][None, :])

  return pl.pallas_call(
      body,
      grid=(_NEB2,),
      in_specs=[_mat_spec(_BE2, F), _full_spec((F, F)), _full_spec((F,))],
      out_specs=_mat_spec(_BE2, F),
      out_shape=jax.ShapeDtypeStruct((N_EDGES, F), f32),
  )(m, mw, mb)


def _tc_bil(x2, sf, bmat, bil_b):
  def body(x_ref, sf_ref, bm_ref, bb_ref, o_ref):
    y = jnp.dot(x_ref[...].astype(jnp.bfloat16), bm_ref[...],
                preferred_element_type=f32)
    sft = sf_ref[...]
    acc = bb_ref[...][None, :]
    for ii in range(8):
      acc = acc + sft[:, ii][:, None] * y[:, ii * F:(ii + 1) * F]
    o_ref[...] = acc

  return pl.pallas_call(
      body,
      grid=(_NEB2,),
      in_specs=[_mat_spec(_BE2, F), _mat_spec(_BE2, 8),
                _full_spec((F, 8 * F)), _full_spec((F,))],
      out_specs=_mat_spec(_BE2, F),
      out_shape=jax.ShapeDtypeStruct((N_TRIPLETS, F), f32),
  )(x2, sf, bmat.astype(jnp.bfloat16), bil_b)


def _tc_update(agg, m, rbf, dw, db, res, rbf_wn, mw_next=None, mb_next=None):
  rp = [res[0]['w1'], res[0]['b1'], res[0]['w2'], res[0]['b2'],
        res[1]['w1'], res[1]['b1'], res[1]['w2'], res[1]['b2']]
  emit_mkj = mw_next is not None

  def body(*refs):
    if emit_mkj:
      (agg_ref, m_ref, rbf_ref, dw_ref, db_ref,
       r0w1, r0b1, r0w2, r0b2, r1w1, r1b1, r1w2, r1b2, rwn_ref,
       mwn_ref, mbn_ref, mn_ref, tn_ref, mkj_ref) = refs
    else:
      (agg_ref, m_ref, rbf_ref, dw_ref, db_ref,
       r0w1, r0b1, r0w2, r0b2, r1w1, r1b1, r1w2, r1b2, rwn_ref,
       mn_ref, tn_ref) = refs
    mm = m_ref[...] + _swish(
        jnp.dot(agg_ref[...], dw_ref[...], preferred_element_type=f32)
        + db_ref[...][None, :])
    for (w1, b1, w2, b2) in ((r0w1, r0b1, r0w2, r0b2),
                             (r1w1, r1b1, r1w2, r1b2)):
      h = _swish(jnp.dot(mm, w1[...], preferred_element_type=f32)
                 + b1[...][None, :])
      mm = mm + jnp.dot(h, w2[...], preferred_element_type=f32) + b2[...][None, :]
    mn_ref[...] = mm
    tn_ref[...] = mm * jnp.dot(rbf_ref[...], rwn_ref[...],
                               preferred_element_type=f32)
    if emit_mkj:
      mkj_ref[...] = _swish(
          jnp.dot(mm, mwn_ref[...], preferred_element_type=f32)
          + mbn_ref[...][None, :])

  ins = [_mat_spec(_BE2, F), _mat_spec(_BE2, F), _mat_spec(_BE2, R),
         _full_spec((F, F)), _full_spec((F,)),
         _full_spec((F, F)), _full_spec((F,)),
         _full_spec((F, F)), _full_spec((F,)),
         _full_spec((F, F)), _full_spec((F,)),
         _full_spec((F, F)), _full_spec((F,)),
         _full_spec((R, F))]
  outs = [_mat_spec(_BE2, F), _mat_spec(_BE2, F)]
  oshapes = [jax.ShapeDtypeStruct((N_EDGES, F), f32),
             jax.ShapeDtypeStruct((N_EDGES, F), f32)]
  args = [agg, m, rbf, dw, db] + rp + [rbf_wn]
  if emit_mkj:
    ins += [_full_spec((F, F)), _full_spec((F,))]
    outs.append(_mat_spec(_BE2, F))
    oshapes.append(jax.ShapeDtypeStruct((N_EDGES, F), f32))
    args += [mw_next, mb_next]

  return pl.pallas_call(
      body,
      grid=(_NEB2,),
      in_specs=ins,
      out_specs=outs,
      out_shape=oshapes,
  )(*args)


_BA = 1000
_NAB = N_ATOMS // _BA


def _tc_atom_out(parts, outp):
  ws = []
  for pb in outp:
    ws += [pb['w1'], pb['b1'], pb['w2'], pb['b2']]

  def body(p0_ref, p1_ref, p2_ref,
           w10, b10, w20, b20, w11, b11, w21, b21, w12, b12, w22, b22,
           o_ref):
    @pl.when(pl.program_id(0) == 0)
    def _():
      o_ref[...] = jnp.zeros((1, 1), f32)

    tot = jnp.zeros((), f32)
    for (p_ref, w1, b1, w2, b2) in (
        (p0_ref, w10, b10, w20, b20),
        (p1_ref, w11, b11, w21, b21),
        (p2_ref, w12, b12, w22, b22)):
      af = p_ref[0] + p_ref[1]
      h = _swish(af)
      h2 = _swish(jnp.dot(h, w1[...], preferred_element_type=f32)
                  + b1[...][None, :])
      o = jnp.dot(h2, w2[...], preferred_element_type=f32) + b2[...][None, :]
      tot = tot + jnp.sum(o)
    o_ref[...] = o_ref[...] + tot.reshape(1, 1)

  pspec = pl.BlockSpec((NC, _BA, F), lambda k: (0, k, 0))
  return pl.pallas_call(
      body,
      grid=(_NAB,),
      in_specs=[pspec, pspec, pspec,
                _full_spec((F, F // 2)), _full_spec((F // 2,)),
                _full_spec((F // 2, 1)), _full_spec((1,)),
                _full_spec((F, F // 2)), _full_spec((F // 2,)),
                _full_spec((F // 2, 1)), _full_spec((1,)),
                _full_spec((F, F // 2)), _full_spec((F // 2,)),
                _full_spec((F // 2, 1)), _full_spec((1,))],
      out_specs=_full_spec((1, 1)),
      out_shape=jax.ShapeDtypeStruct((1, 1), f32),
  )(*parts, *ws)


def kernel(z, pos, i, j, idx_kj, idx_ji, params):
  del z
  pxa, pya, pza = pos[:, 0], pos[:, 1], pos[:, 2]

  d2 = _sc_edge_d2(pxa, pya, pza, i, j)
  dot, n1, n2, d2kj = _sc_triplet_geom(pxa, pya, pza, i, j, idx_kj, idx_ji, d2)

  emb = params['emb']
  cvec = (emb['atom'][0] * emb['atom'][0]).astype(f32)
  outp = params['out']
  ib0_mw = params['inter'][0]['m_w']
  ib0_mb = params['inter'][0]['m_b']
  rbf, m, t0, mkj = _tc_edge(d2, cvec, emb['rbf_w'], emb['rbf_b'],
                             outp[0]['rbf_w'], ib0_mw, ib0_mb)

  ib0, ib1 = params['inter'][0], params['inter'][1]
  sf0, sf1 = _tc_triplet(dot, n1, n2, d2kj,
                         ib0['sbf_w'], ib0['sbf_b'], ib1['sbf_w'], ib1['sbf_b'])

  parts = [_sc_scatter_atoms(t0, i)]
  sfs = (sf0, sf1)
  for blk, ib in enumerate((ib0, ib1)):
    x2 = _sc_row_gather(mkj, idx_kj)
    bmat = jnp.transpose(ib['bil_w'], (2, 1, 0)).reshape(F, 8 * F)
    bil = _tc_bil(x2, sfs[blk], bmat, ib['bil_b'])
    agg = _sc_scatter_edges(bil, idx_ji)
    if blk == 0:
      m, tn, mkj = _tc_update(agg, m, rbf, ib['down_w'], ib['down_b'],
                              ib['res'], outp[blk + 1]['rbf_w'],
                              ib1['m_w'], ib1['m_b'])
    else:
      m, tn = _tc_update(agg, m, rbf, ib['down_w'], ib['down_b'],
                         ib['res'], outp[blk + 1]['rbf_w'])
    parts.append(_sc_scatter_atoms(tn, i))

  total = _tc_atom_out(parts, outp)
  return total[0, 0]


# speedup vs baseline: 2.0615x; 1.0091x over previous
import functools

import jax
import jax.numpy as jnp
import numpy as np
from jax import lax
from jax.experimental import pallas as pl
from jax.experimental.pallas import tpu as pltpu
from jax.experimental.pallas import tpu_sc as plsc

N_ATOMS = 10000
N_EDGES = 160000
N_TRIPLETS = 160000
F = 128
R = 6
S = 7
CUTOFF = 5.0
P = 7

NC, NS, L = 2, 16, 16
NW = NC * NS
EPW = N_EDGES // NW
TPW = N_TRIPLETS // NW
TPS = N_TRIPLETS // NS

def _sc_mesh():
  return plsc.VectorSubcoreMesh(
      core_axis_name="c", subcore_axis_name="s", num_cores=NC, num_subcores=NS)

f32 = jnp.float32
i32 = jnp.int32


def _swish(x):
  return x * jax.nn.sigmoid(x)


def _envelope(x):
  p = P
  a = -(p + 1) * (p + 2) / 2.0
  b = p * (p + 2)
  c = -p * (p + 1) / 2.0
  xp0 = x ** (p - 1)
  xp1 = xp0 * x
  xp2 = xp1 * x
  return 1.0 / x + a * xp0 + b * xp1 + c * xp2


def _wid():
  return lax.axis_index("c") * NS + lax.axis_index("s")


def _sc_edge_d2(pxa, pya, pza, i_arr, j_arr):
  @functools.partial(
      pl.kernel,
      out_type=jax.ShapeDtypeStruct((N_EDGES,), f32),
      mesh=_sc_mesh(),
      compiler_params=pltpu.CompilerParams(needs_layout_passes=False),
      scratch_types=[
          pltpu.VMEM((N_ATOMS,), f32),
          pltpu.VMEM((N_ATOMS,), f32),
          pltpu.VMEM((N_ATOMS,), f32),
          pltpu.VMEM((EPW,), i32),
          pltpu.VMEM((EPW,), i32),
          pltpu.VMEM((EPW,), f32),
      ],
  )
  def k(px_hbm, py_hbm, pz_hbm, i_hbm, j_hbm, d2_hbm, px, py, pz, iv, jv, ov):
    w = _wid()
    base = w * EPW
    pltpu.sync_copy(px_hbm, px)
    pltpu.sync_copy(py_hbm, py)
    pltpu.sync_copy(pz_hbm, pz)
    pltpu.sync_copy(i_hbm.at[pl.ds(base, EPW)], iv)
    pltpu.sync_copy(j_hbm.at[pl.ds(base, EPW)], jv)

    def body(g, _):
      st = jnp.minimum(g * L, EPW - L)
      a16 = iv[pl.ds(st, L)]
      b16 = jv[pl.ds(st, L)]
      dx = plsc.load_gather(px, [a16]) - plsc.load_gather(px, [b16])
      dy = plsc.load_gather(py, [a16]) - plsc.load_gather(py, [b16])
      dz = plsc.load_gather(pz, [a16]) - plsc.load_gather(pz, [b16])
      ov[pl.ds(st, L)] = dx * dx + dy * dy + dz * dz
      return 0

    lax.fori_loop(0, (EPW + L - 1) // L, body, 0)
    pltpu.sync_copy(ov, d2_hbm.at[pl.ds(base, EPW)])

  return k(pxa, pya, pza, i_arr, j_arr)


def _sc_triplet_geom(pxa, pya, pza, i_arr, j_arr, idx_kj, idx_ji, d2):
  out_t = tuple(jax.ShapeDtypeStruct((N_TRIPLETS,), f32) for _ in range(4))

  @functools.partial(
      pl.kernel,
      out_type=out_t,
      mesh=_sc_mesh(),
      compiler_params=pltpu.CompilerParams(needs_layout_passes=False),
      scratch_types=[
          pltpu.VMEM((N_ATOMS,), f32),
          pltpu.VMEM((N_ATOMS,), f32),
          pltpu.VMEM((N_ATOMS,), f32),
          pltpu.VMEM((TPW,), i32),
          pltpu.VMEM((TPW,), i32),
          pltpu.VMEM((TPW,), i32),
          pltpu.VMEM((TPW,), i32),
          pltpu.VMEM((TPW,), i32),
          pltpu.VMEM((TPW,), f32),
          pltpu.VMEM((TPW,), f32),
          pltpu.VMEM((TPW,), f32),
          pltpu.VMEM((TPW,), f32),
          pltpu.SemaphoreType.DMA,
      ],
  )
  def k(px_hbm, py_hbm, pz_hbm, i_hbm, j_hbm, kj_hbm, ji_hbm, d2_hbm,
        dot_hbm, n1_hbm, n2_hbm, d2kj_hbm,
        px, py, pz, jiv, kjv, av, bv, kv, dkv, odot, on1, on2, sem):
    w = _wid()
    base = w * TPW
    pltpu.sync_copy(px_hbm, px)
    pltpu.sync_copy(py_hbm, py)
    pltpu.sync_copy(pz_hbm, pz)
    pltpu.sync_copy(ji_hbm.at[pl.ds(base, TPW)], jiv)
    pltpu.sync_copy(kj_hbm.at[pl.ds(base, TPW)], kjv)
    pltpu.async_copy(i_hbm.at[jiv], av, sem).wait()
    pltpu.async_copy(j_hbm.at[jiv], bv, sem).wait()
    pltpu.async_copy(j_hbm.at[kjv], kv, sem).wait()
    pltpu.async_copy(d2_hbm.at[kjv], dkv, sem).wait()

    def body(g, _):
      st = jnp.minimum(g * L, TPW - L)
      a16 = av[pl.ds(st, L)]
      b16 = bv[pl.ds(st, L)]
      k16 = kv[pl.ds(st, L)]
      bx = plsc.load_gather(px, [b16])
      by = plsc.load_gather(py, [b16])
      bz = plsc.load_gather(pz, [b16])
      v1x = plsc.load_gather(px, [a16]) - bx
      v1y = plsc.load_gather(py, [a16]) - by
      v1z = plsc.load_gather(pz, [a16]) - bz
      v2x = plsc.load_gather(px, [k16]) - bx
      v2y = plsc.load_gather(py, [k16]) - by
      v2z = plsc.load_gather(pz, [k16]) - bz
      odot[pl.ds(st, L)] = v1x * v2x + v1y * v2y + v1z * v2z
      on1[pl.ds(st, L)] = v1x * v1x + v1y * v1y + v1z * v1z
      on2[pl.ds(st, L)] = v2x * v2x + v2y * v2y + v2z * v2z
      return 0

    lax.fori_loop(0, (TPW + L - 1) // L, body, 0)
    pltpu.sync_copy(odot, dot_hbm.at[pl.ds(base, TPW)])
    pltpu.sync_copy(on1, n1_hbm.at[pl.ds(base, TPW)])
    pltpu.sync_copy(on2, n2_hbm.at[pl.ds(base, TPW)])
    pltpu.sync_copy(dkv, d2kj_hbm.at[pl.ds(base, TPW)])

  return k(pxa, pya, pza, i_arr, j_arr, idx_kj, idx_ji, d2)


_G_CHUNK = 200


def _sc_row_gather(tab, idx):
  n_chunks = TPW // _G_CHUNK
  dt = tab.dtype
  dcols = tab.shape[1]

  @functools.partial(
      pl.kernel,
      out_type=jax.ShapeDtypeStruct((N_TRIPLETS, dcols), dt),
      mesh=_sc_mesh(),
      compiler_params=pltpu.CompilerParams(needs_layout_passes=False),
      scratch_types=[
          pltpu.VMEM((_G_CHUNK,), i32),
          pltpu.VMEM((_G_CHUNK, dcols), dt),
          pltpu.SemaphoreType.DMA,
      ],
  )
  def k(tab_hbm, idx_hbm, out_hbm, idxv, rows, sem):
    w = _wid()
    base = w * TPW

    def body(ci, _):
      off = pl.multiple_of(base + ci * _G_CHUNK, 8)
      pltpu.sync_copy(idx_hbm.at[pl.ds(off, _G_CHUNK)], idxv)
      pltpu.async_copy(tab_hbm.at[idxv], rows, sem).wait()
      pltpu.sync_copy(rows, out_hbm.at[pl.ds(off, _G_CHUNK)])
      return 0

    lax.fori_loop(0, n_chunks, body, 0)

  return k(tab, idx)


_CH = 8000
_CHP = 8192
_EPC = N_EDGES // NC
_NCHUNK = _EPC // _CH
_ZROWS = _CHP // NS
_WROWS = 1000


def _sc_scatter_edges(rows_hbm_arr, idx):
  @functools.partial(
      pl.kernel,
      out_type=jax.ShapeDtypeStruct((N_EDGES, F), f32),
      mesh=_sc_mesh(),
      compiler_params=pltpu.CompilerParams(needs_layout_passes=False),
      scratch_types=[
          pltpu.VMEM((TPS,), i32),
          pltpu.VMEM((TPS + 9 * L,), i32),
          pltpu.VMEM((TPS + 9 * L,), i32),
          pltpu.VMEM((4 * L, F), f32),
          pltpu.VMEM((L, F), f32),
          pltpu.VMEM_SHARED((_CHP, F), f32),
          pltpu.SemaphoreType.DMA,
          pltpu.SemaphoreType.DMA,
      ],
  )
  def k(rows_hbm, idx_hbm, agg_hbm, idxv, tlist, llist, rbuf, zrow, acc,
        sem, sem2):
    c = lax.axis_index("c")
    s = lax.axis_index("s")
    sbase = s * TPS
    pltpu.sync_copy(idx_hbm.at[pl.ds(sbase, TPS)], idxv)

    def zfill(r, _):
      for gg in range(F // L):
        zrow[r, pl.ds(gg * L, L)] = jnp.zeros((L,), f32)
      return 0

    lax.fori_loop(0, L, zfill, 0)

    def chunk_body(ci, _):
      cbase = c * _EPC + ci * _CH
      def zcopy(r, _):
        pltpu.sync_copy(
            zrow, acc.at[pl.ds(pl.multiple_of(s * _ZROWS + r * L, 8), L)])
        return 0

      lax.fori_loop(0, _ZROWS // L, zcopy, 0)
      plsc.subcore_barrier()

      def scan_body(g, ptr):
        v = idxv[pl.ds(g * L, L)]
        local = v - cbase
        m = (local >= 0) & (local < _CH)
        tglob = lax.iota(i32, L) + (sbase + g * L)
        plsc.store_compressed(tlist.at[pl.ds(ptr, L)], tglob, mask=m)
        plsc.store_compressed(llist.at[pl.ds(ptr, L)], local, mask=m)
        return ptr + jnp.sum(jnp.where(m, 1, 0).astype(i32))

      ptr = lax.fori_loop(0, TPS // L, scan_body, jnp.int32(0))
      for kk in range(8):
        tlist[pl.ds(ptr + kk * L, L)] = jnp.zeros((L,), i32)
        llist[pl.ds(ptr + kk * L, L)] = jnp.full((L,), _CH, i32)

      nb = (ptr + (L - 1)) // L

      def batch_body(b, _):
        tvec = tlist[pl.ds(b * L, L)]
        lvec = llist[pl.ds(b * L, L)]
        pltpu.async_copy(rows_hbm.at[tvec], rbuf.at[pl.ds(0, L)], sem).wait()
        pltpu.sync_copy(rbuf.at[pl.ds(0, L)], acc.at[lvec], add=True)
        return 0

      lax.fori_loop(0, nb, batch_body, 0)
      plsc.subcore_barrier()
      @pl.when(s < 8)
      def _():
        pltpu.sync_copy(
            acc.at[pl.ds(pl.multiple_of(s * _WROWS, 8), _WROWS)],
            agg_hbm.at[pl.ds(pl.multiple_of(cbase + s * _WROWS, 8), _WROWS)])
      plsc.subcore_barrier()
      return 0

    lax.fori_loop(0, _NCHUNK, chunk_body, 0)

  return k(rows_hbm_arr, idx)


_NP_PAD = 10240
_AZROWS = _NP_PAD // NS


def _sc_scatter_atoms(rows_hbm_arr, i_arr):
  n_full = EPW // L
  tail_valid = EPW - n_full * L

  @functools.partial(
      pl.kernel,
      out_type=jax.ShapeDtypeStruct((NC, _NP_PAD, F), f32),
      mesh=_sc_mesh(),
      compiler_params=pltpu.CompilerParams(needs_layout_passes=False),
      scratch_types=[
          pltpu.VMEM((EPW,), i32),
          pltpu.VMEM((4 * L, F), f32),
          pltpu.VMEM((4 * L, F), f32),
          pltpu.VMEM((L, F), f32),
          pltpu.VMEM_SHARED((_NP_PAD, F), f32),
          pltpu.SemaphoreType.DMA,
          pltpu.SemaphoreType.DMA,
          pltpu.SemaphoreType.DMA,
      ],
  )
  def k(rows_hbm, i_hbm, part_hbm, iv, rbufa, rbufb, zrow, acc,
        sema, semb, sems):
    c = lax.axis_index("c")
    s = lax.axis_index("s")
    w = c * NS + s
    base = w * EPW
    pltpu.sync_copy(i_hbm.at[pl.ds(base, EPW)], iv)

    def zfill(r, _):
      for gg in range(F // L):
        zrow[r, pl.ds(gg * L, L)] = jnp.zeros((L,), f32)
      return 0

    lax.fori_loop(0, L, zfill, 0)

    def zcopy(r, _):
      pltpu.sync_copy(
          zrow, acc.at[pl.ds(pl.multiple_of(s * _AZROWS + r * L, 8), L)])
      return 0

    lax.fori_loop(0, _AZROWS // L, zcopy, 0)
    plsc.subcore_barrier()

    lane = lax.iota(i32, L)
    n_pair = n_full // 8

    def body(m, _):
      st_a = m * (8 * L)
      st_b = st_a + 4 * L
      lda = pltpu.async_copy(
          rows_hbm.at[pl.ds(pl.multiple_of(base + st_a, 8), 4 * L)],
          rbufa, sema)
      ldb = pltpu.async_copy(
          rows_hbm.at[pl.ds(pl.multiple_of(base + st_b, 8), 4 * L)],
          rbufb, semb)
      sds = []
      lda.wait()
      for kk in range(4):
        ivec = iv[pl.ds(st_a + kk * L, L)]
        sds.append(pltpu.async_copy(
            rbufa.at[pl.ds(kk * L, L)], acc.at[ivec], sems, add=True))
      ldb.wait()
      for kk in range(4):
        ivec = iv[pl.ds(st_b + kk * L, L)]
        sds.append(pltpu.async_copy(
            rbufb.at[pl.ds(kk * L, L)], acc.at[ivec], sems, add=True))
      for d in sds:
        d.wait()
      return 0

    lax.fori_loop(0, n_pair, body, 0)
    for g in range(n_pair * 8, n_full):
      st = g * L
      pltpu.sync_copy(
          rows_hbm.at[pl.ds(pl.multiple_of(base + st, 8), L)],
          rbufa.at[pl.ds(0, L)])
      ivec = iv[pl.ds(st, L)]
      pltpu.sync_copy(rbufa.at[pl.ds(0, L)], acc.at[ivec], add=True)
    if tail_valid:
      st = EPW - L
      pltpu.sync_copy(rows_hbm.at[pl.ds(pl.multiple_of(base + st, 8), L)],
                      rbufa.at[pl.ds(0, L)])
      ivec = iv[pl.ds(st, L)]
      ivec = jnp.where(lane < (L - tail_valid), jnp.full((L,), N_ATOMS, i32),
                       ivec)
      pltpu.sync_copy(rbufa.at[pl.ds(0, L)], acc.at[ivec], add=True)

    plsc.subcore_barrier()
    pltpu.sync_copy(
        acc.at[pl.ds(pl.multiple_of(s * _AZROWS, 8), _AZROWS)],
        part_hbm.at[c, pl.ds(pl.multiple_of(s * _AZROWS, 8), _AZROWS)])

  return k(rows_hbm_arr, i_arr)


_BE = 1024
_NEB = (N_EDGES + _BE - 1) // _BE
_BE2 = 4000
_NEB2 = N_EDGES // _BE2

def _freq_row():
  return (lax.broadcasted_iota(jnp.int32, (1, R), 1).astype(f32) + 1.0) * np.pi


def _sbf_consts():
  q = lax.broadcasted_iota(jnp.int32, (1, S * R), 1)
  r = (q % R).astype(f32)
  s = (q // R).astype(f32)
  bwf = (r + 1.0) * np.pi + s * (S - 1) * np.pi
  return bwf, s


def _vec_spec(b):
  return pl.BlockSpec((b,), lambda k: (k,))


def _mat_spec(b, d):
  return pl.BlockSpec((b, d), lambda k: (k, 0))


def _full_spec(shape):
  nd = len(shape)
  return pl.BlockSpec(shape, lambda k: (0,) * nd)


def _tc_edge(d2, cvec, emb_w, emb_b, rbf_w0, mw0, mb0):
  def body(d2_ref, c_ref, ew_ref, eb_ref, rw0_ref, mw_ref, mb_ref,
           rbf_ref, m0_ref, t0_ref, mkj_ref):
    d2t = d2_ref[...]
    dist = jnp.sqrt(d2t)
    ds_ = dist / CUTOFF
    env = _envelope(ds_)
    rbf = env[:, None] * jnp.sin(_freq_row() * ds_[:, None]) / dist[:, None]
    rbf_ref[...] = rbf
    m0 = c_ref[...][None, :] * _swish(
        jnp.dot(rbf, ew_ref[...], preferred_element_type=f32)
        + eb_ref[...][None, :])
    m0_ref[...] = m0
    t0_ref[...] = m0 * jnp.dot(rbf, rw0_ref[...], preferred_element_type=f32)
    mkj_ref[...] = _swish(
        jnp.dot(m0, mw_ref[...], preferred_element_type=f32)
        + mb_ref[...][None, :])

  return pl.pallas_call(
      body,
      grid=(_NEB,),
      in_specs=[_vec_spec(_BE), _full_spec((F,)), _full_spec((R, F)),
                _full_spec((F,)), _full_spec((R, F)),
                _full_spec((F, F)), _full_spec((F,))],
      out_specs=[_mat_spec(_BE, R), _mat_spec(_BE, F), _mat_spec(_BE, F),
                 _mat_spec(_BE, F)],
      out_shape=[jax.ShapeDtypeStruct((N_EDGES, R), f32),
                 jax.ShapeDtypeStruct((N_EDGES, F), f32),
                 jax.ShapeDtypeStruct((N_EDGES, F), f32),
                 jax.ShapeDtypeStruct((N_EDGES, F), f32)],
  )(d2, cvec, emb_w, emb_b, rbf_w0, mw0, mb0)


def _tc_triplet(dot, n1, n2, d2kj, sw0, sb0, sw1, sb1):
  def body(dot_ref, n1_ref, n2_ref, dk_ref, w0_ref, b0_ref, w1_ref, b1_ref,
           sf0_ref, sf1_ref):
    dott = dot_ref[...]
    nrm = jnp.sqrt(n1_ref[...] + 1e-12) * jnp.sqrt(n2_ref[...] + 1e-12)
    ca = jnp.clip(dott / (nrm + 1e-7), -1.0 + 1e-6, 1.0 - 1e-6)
    dk = jnp.sqrt(dk_ref[...]) / CUTOFF
    env = _envelope(dk)
    bwf, shwf = _sbf_consts()
    cprev = jnp.ones_like(ca)
    ccur = ca
    sph = (shwf == 0.0).astype(f32) * cprev[:, None]
    sph = sph + (shwf == 1.0).astype(f32) * ccur[:, None]
    for s_ in range(2, S):
      cnext = 2.0 * ca * ccur - cprev
      cprev, ccur = ccur, cnext
      sph = sph + (shwf == float(s_)).astype(f32) * ccur[:, None]
    sbf = env[:, None] * jnp.sin(bwf * dk[:, None]) * sph
    sf0_ref[...] = _swish(
        jnp.dot(sbf, w0_ref[...], preferred_element_type=f32)
        + b0_ref[...][None, :])
    sf1_ref[...] = _swish(
        jnp.dot(sbf, w1_ref[...], preferred_element_type=f32)
        + b1_ref[...][None, :])

  return pl.pallas_call(
      body,
      grid=(_NEB,),
      in_specs=[_vec_spec(_BE)] * 4
      + [_full_spec((S * R, 8)), _full_spec((8,)),
         _full_spec((S * R, 8)), _full_spec((8,))],
      out_specs=[_mat_spec(_BE, 8), _mat_spec(_BE, 8)],
      out_shape=[jax.ShapeDtypeStruct((N_TRIPLETS, 8), f32),
                 jax.ShapeDtypeStruct((N_TRIPLETS, 8), f32)],
  )(dot, n1, n2, d2kj, sw0, sb0, sw1, sb1)


def _tc_mkj(m, mw, mb):
  def body(m_ref, w_ref, b_ref, o_ref):
    o_ref[...] = _swish(
        jnp.dot(m_ref[...], w_ref[...], preferred_element_type=f32)
        + b_ref[...][None, :])

  return pl.pallas_call(
      body,
      grid=(_NEB2,),
      in_specs=[_mat_spec(_BE2, F), _full_spec((F, F)), _full_spec((F,))],
      out_specs=_mat_spec(_BE2, F),
      out_shape=jax.ShapeDtypeStruct((N_EDGES, F), f32),
  )(m, mw, mb)


def _tc_bil(x2, sf, bmat, bil_b):
  def body(x_ref, sf_ref, bm_ref, bb_ref, o_ref):
    y = jnp.dot(x_ref[...].astype(jnp.bfloat16), bm_ref[...],
                preferred_element_type=f32)
    sft = sf_ref[...]
    acc = bb_ref[...][None, :]
    for ii in range(8):
      acc = acc + sft[:, ii][:, None] * y[:, ii * F:(ii + 1) * F]
    o_ref[...] = acc

  return pl.pallas_call(
      body,
      grid=(_NEB2,),
      in_specs=[_mat_spec(_BE2, F), _mat_spec(_BE2, 8),
                _full_spec((F, 8 * F)), _full_spec((F,))],
      out_specs=_mat_spec(_BE2, F),
      out_shape=jax.ShapeDtypeStruct((N_TRIPLETS, F), f32),
  )(x2, sf, bmat.astype(jnp.bfloat16), bil_b)


def _tc_update(agg, m, rbf, dw, db, res, rbf_wn, mw_next=None, mb_next=None):
  rp = [res[0]['w1'], res[0]['b1'], res[0]['w2'], res[0]['b2'],
        res[1]['w1'], res[1]['b1'], res[1]['w2'], res[1]['b2']]
  emit_mkj = mw_next is not None

  def body(*refs):
    if emit_mkj:
      (agg_ref, m_ref, rbf_ref, dw_ref, db_ref,
       r0w1, r0b1, r0w2, r0b2, r1w1, r1b1, r1w2, r1b2, rwn_ref,
       mwn_ref, mbn_ref, mn_ref, tn_ref, mkj_ref) = refs
    else:
      (agg_ref, m_ref, rbf_ref, dw_ref, db_ref,
       r0w1, r0b1, r0w2, r0b2, r1w1, r1b1, r1w2, r1b2, rwn_ref,
       mn_ref, tn_ref) = refs
    mm = m_ref[...] + _swish(
        jnp.dot(agg_ref[...], dw_ref[...], preferred_element_type=f32)
        + db_ref[...][None, :])
    for (w1, b1, w2, b2) in ((r0w1, r0b1, r0w2, r0b2),
                             (r1w1, r1b1, r1w2, r1b2)):
      h = _swish(jnp.dot(mm, w1[...], preferred_element_type=f32)
                 + b1[...][None, :])
      mm = mm + jnp.dot(h, w2[...], preferred_element_type=f32) + b2[...][None, :]
    mn_ref[...] = mm
    tn_ref[...] = mm * jnp.dot(rbf_ref[...], rwn_ref[...],
                               preferred_element_type=f32)
    if emit_mkj:
      mkj_ref[...] = _swish(
          jnp.dot(mm, mwn_ref[...], preferred_element_type=f32)
          + mbn_ref[...][None, :])

  ins = [_mat_spec(_BE2, F), _mat_spec(_BE2, F), _mat_spec(_BE2, R),
         _full_spec((F, F)), _full_spec((F,)),
         _full_spec((F, F)), _full_spec((F,)),
         _full_spec((F, F)), _full_spec((F,)),
         _full_spec((F, F)), _full_spec((F,)),
         _full_spec((F, F)), _full_spec((F,)),
         _full_spec((R, F))]
  outs = [_mat_spec(_BE2, F), _mat_spec(_BE2, F)]
  oshapes = [jax.ShapeDtypeStruct((N_EDGES, F), f32),
             jax.ShapeDtypeStruct((N_EDGES, F), f32)]
  args = [agg, m, rbf, dw, db] + rp + [rbf_wn]
  if emit_mkj:
    ins += [_full_spec((F, F)), _full_spec((F,))]
    outs.append(_mat_spec(_BE2, F))
    oshapes.append(jax.ShapeDtypeStruct((N_EDGES, F), f32))
    args += [mw_next, mb_next]

  return pl.pallas_call(
      body,
      grid=(_NEB2,),
      in_specs=ins,
      out_specs=outs,
      out_shape=oshapes,
  )(*args)


_BA = 2000
_NAB = N_ATOMS // _BA


def _tc_atom_out(parts, outp):
  ws = []
  for pb in outp:
    ws += [pb['w1'], pb['b1'], pb['w2'], pb['b2']]

  def body(p0_ref, p1_ref, p2_ref,
           w10, b10, w20, b20, w11, b11, w21, b21, w12, b12, w22, b22,
           o_ref):
    @pl.when(pl.program_id(0) == 0)
    def _():
      o_ref[...] = jnp.zeros((1, 1), f32)

    tot = jnp.zeros((), f32)
    for (p_ref, w1, b1, w2, b2) in (
        (p0_ref, w10, b10, w20, b20),
        (p1_ref, w11, b11, w21, b21),
        (p2_ref, w12, b12, w22, b22)):
      af = p_ref[0] + p_ref[1]
      h = _swish(af)
      h2 = _swish(jnp.dot(h, w1[...], preferred_element_type=f32)
                  + b1[...][None, :])
      o = jnp.dot(h2, w2[...], preferred_element_type=f32) + b2[...][None, :]
      tot = tot + jnp.sum(o)
    o_ref[...] = o_ref[...] + tot.reshape(1, 1)

  pspec = pl.BlockSpec((NC, _BA, F), lambda k: (0, k, 0))
  return pl.pallas_call(
      body,
      grid=(_NAB,),
      in_specs=[pspec, pspec, pspec,
                _full_spec((F, F // 2)), _full_spec((F // 2,)),
                _full_spec((F // 2, 1)), _full_spec((1,)),
                _full_spec((F, F // 2)), _full_spec((F // 2,)),
                _full_spec((F // 2, 1)), _full_spec((1,)),
                _full_spec((F, F // 2)), _full_spec((F // 2,)),
                _full_spec((F // 2, 1)), _full_spec((1,))],
      out_specs=_full_spec((1, 1)),
      out_shape=jax.ShapeDtypeStruct((1, 1), f32),
  )(*parts, *ws)


def kernel(z, pos, i, j, idx_kj, idx_ji, params):
  del z
  pxa, pya, pza = pos[:, 0], pos[:, 1], pos[:, 2]

  d2 = _sc_edge_d2(pxa, pya, pza, i, j)
  dot, n1, n2, d2kj = _sc_triplet_geom(pxa, pya, pza, i, j, idx_kj, idx_ji, d2)

  emb = params['emb']
  cvec = (emb['atom'][0] * emb['atom'][0]).astype(f32)
  outp = params['out']
  ib0_mw = params['inter'][0]['m_w']
  ib0_mb = params['inter'][0]['m_b']
  rbf, m, t0, mkj = _tc_edge(d2, cvec, emb['rbf_w'], emb['rbf_b'],
                             outp[0]['rbf_w'], ib0_mw, ib0_mb)

  ib0, ib1 = params['inter'][0], params['inter'][1]
  sf0, sf1 = _tc_triplet(dot, n1, n2, d2kj,
                         ib0['sbf_w'], ib0['sbf_b'], ib1['sbf_w'], ib1['sbf_b'])

  parts = [_sc_scatter_atoms(t0, i)]
  sfs = (sf0, sf1)
  for blk, ib in enumerate((ib0, ib1)):
    x2 = _sc_row_gather(mkj, idx_kj)
    bmat = jnp.transpose(ib['bil_w'], (2, 1, 0)).reshape(F, 8 * F)
    bil = _tc_bil(x2, sfs[blk], bmat, ib['bil_b'])
    agg = _sc_scatter_edges(bil, idx_ji)
    if blk == 0:
      m, tn, mkj = _tc_update(agg, m, rbf, ib['down_w'], ib['down_b'],
                              ib['res'], outp[blk + 1]['rbf_w'],
                              ib1['m_w'], ib1['m_b'])
    else:
      m, tn = _tc_update(agg, m, rbf, ib['down_w'], ib['down_b'],
                         ib['res'], outp[blk + 1]['rbf_w'])
    parts.append(_sc_scatter_atoms(tn, i))

  total = _tc_atom_out(parts, outp)
  return total[0, 0]
